# Initial kernel scaffold; baseline (speedup 1.0000x reference)
import jax
import jax.numpy as jnp
from jax.experimental import pallas as pl


def kernel(ht, r_tensor, entity_feat, relation_feat, p_selections, queries, params):
    raise NotImplementedError("write your pallas kernel here")


# speedup vs baseline: 4.2107x; 4.2107x over previous
import functools

import jax
import jax.numpy as jnp
from jax.experimental import pallas as pl
from jax.experimental.pallas import tpu as pltpu
from jax.experimental.pallas import tpu_sc as plsc

F32 = jnp.float32
LANES = 128
NW = 32
TE = 2000
TN = 2048


def _leaky(x):
    return jnp.where(x >= 0, x, 0.01 * x)


def _lnorm(x, g, b):
    mu = x.mean(axis=-1, keepdims=True)
    var = ((x - mu) ** 2).mean(axis=-1, keepdims=True)
    return (x - mu) / jnp.sqrt(var + 1e-5) * g + b


def _dot(a, b):
    return jnp.dot(a, b, preferred_element_type=F32)


def _full(shape):
    return pl.BlockSpec(shape, lambda i: (0,) * len(shape))


def _node_encoder(x, wt, b, g, bb):
    n = x.shape[0]

    def body(x_ref, wt_ref, b_ref, g_ref, bb_ref, o_ref):
        y = _dot(x_ref[...], wt_ref[...]) + b_ref[...]
        o_ref[...] = _lnorm(_leaky(y), g_ref[...], bb_ref[...])

    return pl.pallas_call(
        body,
        grid=(n // TN,),
        in_specs=[
            pl.BlockSpec((TN, LANES), lambda i: (i, 0)),
            _full((LANES, LANES)), _full((1, LANES)), _full((1, LANES)),
            _full((1, LANES)),
        ],
        out_specs=pl.BlockSpec((TN, LANES), lambda i: (i, 0)),
        out_shape=jax.ShapeDtypeStruct((n, LANES), F32),
    )(x, wt, b, g, bb)


def _rel_table(relation_feat, w128t, rel_embed):
    nr = relation_feat.shape[0]

    def body(rf_ref, wt_ref, re_ref, o_ref):
        o_ref[:, 0:LANES] = _dot(rf_ref[...], wt_ref[...])
        o_ref[:, LANES:2 * LANES] = re_ref[...]

    return pl.pallas_call(
        body,
        grid=(1,),
        in_specs=[_full((nr, LANES)), _full((LANES, LANES)),
                  _full((nr, LANES))],
        out_specs=_full((nr, 2 * LANES)),
        out_shape=jax.ShapeDtypeStruct((nr, 2 * LANES), F32),
    )(relation_feat, w128t, rel_embed)


def _edge_encoder(t0g, q, wq, b, g, bb):
    m = t0g.shape[0]

    def body(r1_ref, q_ref, wq_ref, b_ref, g_ref, bb_ref, o_ref):
        y = r1_ref[...] + q_ref[...] * wq_ref[...] + b_ref[...]
        o_ref[...] = _lnorm(_leaky(y), g_ref[...], bb_ref[...])

    return pl.pallas_call(
        body,
        grid=(m // TE,),
        in_specs=[
            pl.BlockSpec((TE, LANES), lambda i: (i, 0)),
            pl.BlockSpec((TE, 1), lambda i: (i, 0)),
            _full((1, LANES)), _full((1, LANES)), _full((1, LANES)),
            _full((1, LANES)),
        ],
        out_specs=pl.BlockSpec((TE, LANES), lambda i: (i, 0)),
        out_shape=jax.ShapeDtypeStruct((m, LANES), F32),
    )(t0g, q, wq, b, g, bb)


def _msg(hh, ht, e, t0g, wft, wbt, bf, bb):
    m = e.shape[0]
    L = LANES

    def body(hh_ref, ht_ref, e_ref, r_ref, wf_ref, wb_ref, bf_ref, bb_ref,
             mf_ref, mb_ref):
        ev = e_ref[...]
        rv = r_ref[...]
        er = ev * rv
        wf = wf_ref[...]
        wb = wb_ref[...]
        hhv = hh_ref[...]
        htv = ht_ref[...]
        mf_ref[...] = (_dot(hhv, wf[0:L]) + _dot(ev, wf[L:2 * L])
                       + _dot(hhv * rv, wf[2 * L:3 * L])
                       + _dot(er, wf[3 * L:4 * L]) + bf_ref[...])
        mb_ref[...] = (_dot(htv, wb[0:L]) + _dot(ev, wb[L:2 * L])
                       + _dot(htv * rv, wb[2 * L:3 * L])
                       + _dot(er, wb[3 * L:4 * L]) + bb_ref[...])

    edge = pl.BlockSpec((TE, LANES), lambda i: (i, 0))
    return pl.pallas_call(
        body,
        grid=(m // TE,),
        in_specs=[
            edge, edge, edge,
            pl.BlockSpec((TE, LANES), lambda i: (i, 1)),
            _full((4 * LANES, LANES)), _full((4 * LANES, LANES)),
            _full((1, LANES)), _full((1, LANES)),
        ],
        out_specs=[edge, edge],
        out_shape=[jax.ShapeDtypeStruct((m, LANES), F32),
                   jax.ShapeDtypeStruct((m, LANES), F32)],
    )(hh, ht, e, t0g, wft, wbt, bf, bb)


def _h_update(partials, cnt_partials, h, g, b):
    n = h.shape[0]

    def body(p_ref, c_ref, h_ref, g_ref, b_ref, o_ref):
        agg = p_ref[0] + p_ref[1]
        cnt = (c_ref[0] + c_ref[1])[:, None]
        o_ref[...] = _lnorm(_leaky(agg / cnt) + h_ref[...],
                            g_ref[...], b_ref[...])

    return pl.pallas_call(
        body,
        grid=(n // TN,),
        in_specs=[
            pl.BlockSpec((2, TN, LANES), lambda i: (0, i, 0)),
            pl.BlockSpec((2, TN), lambda i: (0, i)),
            pl.BlockSpec((TN, LANES), lambda i: (i, 0)),
            _full((1, LANES)), _full((1, LANES)),
        ],
        out_specs=pl.BlockSpec((TN, LANES), lambda i: (i, 0)),
        out_shape=jax.ShapeDtypeStruct((n, LANES), F32),
    )(partials, cnt_partials, h, g, b)


def _edge_update(hh, ht, e, wt, b, g, bb):
    m = e.shape[0]
    L = LANES

    def body(hh_ref, ht_ref, e_ref, w_ref, b_ref, g_ref, bb_ref, o_ref):
        ev = e_ref[...]
        w = w_ref[...]
        y = (_dot(hh_ref[...], w[0:L]) + _dot(ev, w[L:2 * L])
             + _dot(ht_ref[...], w[2 * L:3 * L]) + b_ref[...])
        o_ref[...] = _lnorm(_leaky(y) + ev, g_ref[...], bb_ref[...])

    edge = pl.BlockSpec((TE, LANES), lambda i: (i, 0))
    return pl.pallas_call(
        body,
        grid=(m // TE,),
        in_specs=[edge, edge, edge, _full((3 * LANES, LANES)),
                  _full((1, LANES)), _full((1, LANES)), _full((1, LANES))],
        out_specs=edge,
        out_shape=jax.ShapeDtypeStruct((m, LANES), F32),
    )(hh, ht, e, wt, b, g, bb)


def _classifier(e2, e0, hh, h0h, htl, h0t, w1t, b1, w2, b2):
    m = e2.shape[0]
    L = LANES

    def body(e2_ref, e0_ref, hh_ref, h0h_ref, ht_ref, h0t_ref,
             w_ref, b1_ref, w2_ref, b2_ref, o_ref):
        w = w_ref[...]
        y = (_dot(e2_ref[...], w[0:L]) + _dot(e0_ref[...], w[L:2 * L])
             + _dot(hh_ref[...], w[2 * L:3 * L])
             + _dot(h0h_ref[...], w[3 * L:4 * L])
             + _dot(ht_ref[...], w[4 * L:5 * L])
             + _dot(h0t_ref[...], w[5 * L:6 * L]) + b1_ref[...])
        o1 = _leaky(y)
        o_ref[...] = (jnp.sum(o1 * w2_ref[...], axis=1, keepdims=True)
                      + b2_ref[...])

    edge = pl.BlockSpec((TE, LANES), lambda i: (i, 0))
    return pl.pallas_call(
        body,
        grid=(m // TE,),
        in_specs=[edge, edge, edge, edge, edge, edge,
                  _full((6 * LANES, LANES)), _full((1, LANES)),
                  _full((1, LANES)), _full((1, 1))],
        out_specs=pl.BlockSpec((TE, 1), lambda i: (i, 0)),
        out_shape=jax.ShapeDtypeStruct((m, 1), F32),
    )(e2, e0, hh, h0h, htl, h0t, w1t, b1, w2, b2)


def _gather_rows(table, idx):
    n, d = table.shape
    r = idx.shape[0] // 128
    base, rem = r // NW, r % NW

    def body(tab_hbm, idx_hbm, out_hbm, idx_v, rows_v, sem):
        c = jax.lax.axis_index("c")
        s = jax.lax.axis_index("s")
        w = s * 2 + c
        nrows = base + jnp.where(w < rem, 1, 0)
        start = w * base + jnp.minimum(w, rem)

        def step(j, carry):
            @pl.when(j < nrows)
            def _():
                cid = start + j
                pltpu.sync_copy(idx_hbm.at[pl.ds(cid * 128, 128)], idx_v)
                pltpu.async_copy(tab_hbm.at[idx_v], rows_v, sem).wait()
                pltpu.sync_copy(rows_v, out_hbm.at[pl.ds(cid * 128, 128)])
            return carry

        jax.lax.fori_loop(0, base + 1, step, 0)

    f = pl.kernel(
        body,
        out_type=jax.ShapeDtypeStruct((r * 128, d), F32),
        mesh=plsc.VectorSubcoreMesh(core_axis_name="c", subcore_axis_name="s"),
        scratch_types=[
            pltpu.VMEM((128,), jnp.int32),
            pltpu.VMEM((128, d), F32),
            pltpu.SemaphoreType.DMA,
        ],
    )
    return f(table, idx)


def _scatter_add(mf, mb, idxf, idxb, n):
    rh = idxf.shape[0] // 128 // 2
    base, rem = rh // 16, rh % 16
    npad = ((n + 2047) // 2048) * 2048
    npt = npad // 16

    def body(mf_hbm, mb_hbm, if_hbm, ib_hbm, outp_hbm, outc_hbm,
             tab_sh, cnt_sh, idx_v, rows_v, zb_v, ones_v, cbuf_v, sem):
        c = jax.lax.axis_index("c")
        s = jax.lax.axis_index("s")
        zv = jnp.zeros((16,), F32)
        ov = jnp.ones((16,), F32)

        def zfill(i, carry):
            zb_v[i // 8, pl.ds((i % 8) * 16, 16)] = zv
            return carry
        jax.lax.fori_loop(0, 128 * 8, zfill, 0)

        def ofill(i, carry):
            ones_v[pl.ds(i * 16, 16)] = ov
            cbuf_v[pl.ds(i * 16, 16)] = zv
            return carry
        jax.lax.fori_loop(0, 8, ofill, 0)

        def ztab(z, carry):
            pltpu.sync_copy(zb_v, tab_sh.at[pl.ds(s * npt + z * 128, 128)])
            pltpu.sync_copy(cbuf_v, cnt_sh.at[pl.ds(s * npt + z * 128, 128)])
            return carry
        jax.lax.fori_loop(0, npt // 128, ztab, 0)
        plsc.subcore_barrier()

        for msg_hbm, idx_hbm in ((mf_hbm, if_hbm), (mb_hbm, ib_hbm)):
            nrows = base + jnp.where(s < rem, 1, 0)
            start = c * rh + s * base + jnp.minimum(s, rem)

            def step(j, carry):
                @pl.when(j < nrows)
                def _():
                    cid = start + j
                    pltpu.sync_copy(idx_hbm.at[pl.ds(cid * 128, 128)], idx_v)
                    pltpu.sync_copy(msg_hbm.at[pl.ds(cid * 128, 128)], rows_v)
                    pltpu.sync_copy(rows_v, tab_sh.at[idx_v], add=True)
                    pltpu.sync_copy(ones_v, cnt_sh.at[idx_v], add=True)
                return carry
            jax.lax.fori_loop(0, base + 1, step, 0)
        plsc.subcore_barrier()

        def wout(z, carry):
            a = s * npt + z * 128
            pltpu.sync_copy(tab_sh.at[pl.ds(a, 128)], rows_v)
            pltpu.sync_copy(rows_v, outp_hbm.at[c, pl.ds(a, 128)])
            pltpu.sync_copy(cnt_sh.at[pl.ds(a, 128)], cbuf_v)
            pltpu.sync_copy(cbuf_v, outc_hbm.at[c, pl.ds(a, 128)])
            return carry
        jax.lax.fori_loop(0, npt // 128, wout, 0)

    f = pl.kernel(
        body,
        out_type=(jax.ShapeDtypeStruct((2, npad, LANES), F32),
                  jax.ShapeDtypeStruct((2, npad), F32)),
        mesh=plsc.VectorSubcoreMesh(core_axis_name="c", subcore_axis_name="s"),
        scratch_types=[
            pltpu.VMEM_SHARED((npad, LANES), F32),
            pltpu.VMEM_SHARED((npad,), F32),
            pltpu.VMEM((128,), jnp.int32),
            pltpu.VMEM((128, LANES), F32),
            pltpu.VMEM((128, 128), F32),
            pltpu.VMEM((128,), F32),
            pltpu.VMEM((128,), F32),
            pltpu.SemaphoreType.DMA,
        ],
    )
    return f(mf, mb, idxf, idxb)


def kernel(ht, r_tensor, entity_feat, relation_feat, p_selections, queries,
           params):
    del p_selections
    m = ht.shape[0]
    n = entity_feat.shape[0]
    npad = ((n + 2047) // 2048) * 2048
    ef_pad = jnp.pad(entity_feat, ((0, npad - n), (0, 0)))

    hix = ht[:, 0].astype(jnp.int32)
    tix = ht[:, 1].astype(jnp.int32)
    rix = r_tensor.astype(jnp.int32)
    q2d = queries.reshape(m, 1)

    p = params
    row = lambda v: v.reshape(1, -1)

    h0 = _node_encoder(ef_pad, p['ent_in_W'].T, row(p['ent_in_b']),
                       row(p['ln_ent_g']), row(p['ln_ent_b']))
    t0 = _rel_table(relation_feat, p['edge_in_W'][:, :LANES].T,
                    p['rel_embed'])
    t0g = _gather_rows(t0, rix)
    e0 = _edge_encoder(t0g, q2d, row(p['edge_in_W'][:, LANES]),
                       row(p['edge_in_b']), row(p['ln_edge_g']),
                       row(p['ln_edge_b']))

    h0h = _gather_rows(h0, hix)
    h0t = _gather_rows(h0, tix)

    h, e = h0, e0
    hh, htl = h0h, h0t
    for lp in p['layers']:
        mfv, mbv = _msg(hh, htl, e, t0g, lp['fwd_W'].T, lp['back_W'].T,
                        row(lp['fwd_b']), row(lp['back_b']))
        partials, cnt_partials = _scatter_add(mfv, mbv, tix, hix, npad)
        h = _h_update(partials, cnt_partials, h, row(lp['mp_ln_g']),
                      row(lp['mp_ln_b']))
        hh = _gather_rows(h, hix)
        htl = _gather_rows(h, tix)
        e = _edge_update(hh, htl, e, lp['eu_W'].T, row(lp['eu_b']),
                         row(lp['eu_ln_g']), row(lp['eu_ln_b']))

    return _classifier(e, e0, hh, h0h, htl, h0t, p['cls_W1'].T,
                       row(p['cls_b1']), row(p['cls_W2']),
                       p['cls_b2'].reshape(1, 1))


# speedup vs baseline: 5.1085x; 1.2132x over previous
import functools

import jax
import jax.numpy as jnp
from jax.experimental import pallas as pl
from jax.experimental.pallas import tpu as pltpu
from jax.experimental.pallas import tpu_sc as plsc

F32 = jnp.float32
LANES = 128
NW = 32
TE = 2000
TN = 2048


def _leaky(x):
    return jnp.where(x >= 0, x, 0.01 * x)


def _lnorm(x, g, b):
    mu = x.mean(axis=-1, keepdims=True)
    var = ((x - mu) ** 2).mean(axis=-1, keepdims=True)
    return (x - mu) / jnp.sqrt(var + 1e-5) * g + b


def _dot(a, b):
    return jnp.dot(a, b, preferred_element_type=F32)


def _full(shape):
    return pl.BlockSpec(shape, lambda i: (0,) * len(shape))


def _node_encoder(x, wt, b, g, bb):
    n = x.shape[0]

    def body(x_ref, wt_ref, b_ref, g_ref, bb_ref, o_ref):
        y = _dot(x_ref[...], wt_ref[...]) + b_ref[...]
        o_ref[...] = _lnorm(_leaky(y), g_ref[...], bb_ref[...])

    return pl.pallas_call(
        body,
        grid=(n // TN,),
        in_specs=[
            pl.BlockSpec((TN, LANES), lambda i: (i, 0)),
            _full((LANES, LANES)), _full((1, LANES)), _full((1, LANES)),
            _full((1, LANES)),
        ],
        out_specs=pl.BlockSpec((TN, LANES), lambda i: (i, 0)),
        out_shape=jax.ShapeDtypeStruct((n, LANES), F32),
    )(x, wt, b, g, bb)


def _rel_table(relation_feat, w128t, rel_embed):
    nr = relation_feat.shape[0]

    def body(rf_ref, wt_ref, re_ref, o_ref):
        o_ref[:, 0:LANES] = _dot(rf_ref[...], wt_ref[...])
        o_ref[:, LANES:2 * LANES] = re_ref[...]

    return pl.pallas_call(
        body,
        grid=(1,),
        in_specs=[_full((nr, LANES)), _full((LANES, LANES)),
                  _full((nr, LANES))],
        out_specs=_full((nr, 2 * LANES)),
        out_shape=jax.ShapeDtypeStruct((nr, 2 * LANES), F32),
    )(relation_feat, w128t, rel_embed)


def _edge_encoder(t0g, q, wq, b, g, bb):
    m = t0g.shape[0]

    def body(r1_ref, q_ref, wq_ref, b_ref, g_ref, bb_ref, o_ref):
        y = r1_ref[...] + q_ref[...] * wq_ref[...] + b_ref[...]
        o_ref[...] = _lnorm(_leaky(y), g_ref[...], bb_ref[...])

    return pl.pallas_call(
        body,
        grid=(m // TE,),
        in_specs=[
            pl.BlockSpec((TE, LANES), lambda i: (i, 0)),
            pl.BlockSpec((TE, 1), lambda i: (i, 0)),
            _full((1, LANES)), _full((1, LANES)), _full((1, LANES)),
            _full((1, LANES)),
        ],
        out_specs=pl.BlockSpec((TE, LANES), lambda i: (i, 0)),
        out_shape=jax.ShapeDtypeStruct((m, LANES), F32),
    )(t0g, q, wq, b, g, bb)


def _msg(hh, ht, e, t0g, wft, wbt, bf, bb):
    m = e.shape[0]
    L = LANES

    def body(hh_ref, ht_ref, e_ref, r_ref, wf_ref, wb_ref, bf_ref, bb_ref,
             mf_ref, mb_ref):
        ev = e_ref[...]
        rv = r_ref[...]
        er = ev * rv
        wf = wf_ref[...]
        wb = wb_ref[...]
        hhv = hh_ref[...]
        htv = ht_ref[...]
        mf_ref[...] = (_dot(hhv, wf[0:L]) + _dot(ev, wf[L:2 * L])
                       + _dot(hhv * rv, wf[2 * L:3 * L])
                       + _dot(er, wf[3 * L:4 * L]) + bf_ref[...])
        mb_ref[...] = (_dot(htv, wb[0:L]) + _dot(ev, wb[L:2 * L])
                       + _dot(htv * rv, wb[2 * L:3 * L])
                       + _dot(er, wb[3 * L:4 * L]) + bb_ref[...])

    edge = pl.BlockSpec((TE, LANES), lambda i: (i, 0))
    return pl.pallas_call(
        body,
        grid=(m // TE,),
        in_specs=[
            edge, edge, edge,
            pl.BlockSpec((TE, LANES), lambda i: (i, 1)),
            _full((4 * LANES, LANES)), _full((4 * LANES, LANES)),
            _full((1, LANES)), _full((1, LANES)),
        ],
        out_specs=[edge, edge],
        out_shape=[jax.ShapeDtypeStruct((m, LANES), F32),
                   jax.ShapeDtypeStruct((m, LANES), F32)],
    )(hh, ht, e, t0g, wft, wbt, bf, bb)


def _h_update(partials, cnt_partials, h, g, b):
    n = h.shape[0]

    def body(p_ref, c_ref, h_ref, g_ref, b_ref, o_ref):
        agg = p_ref[0] + p_ref[1]
        cnt = (c_ref[0] + c_ref[1])[:, None]
        o_ref[...] = _lnorm(_leaky(agg / cnt) + h_ref[...],
                            g_ref[...], b_ref[...])

    return pl.pallas_call(
        body,
        grid=(n // TN,),
        in_specs=[
            pl.BlockSpec((2, TN, LANES), lambda i: (0, i, 0)),
            pl.BlockSpec((2, TN), lambda i: (0, i)),
            pl.BlockSpec((TN, LANES), lambda i: (i, 0)),
            _full((1, LANES)), _full((1, LANES)),
        ],
        out_specs=pl.BlockSpec((TN, LANES), lambda i: (i, 0)),
        out_shape=jax.ShapeDtypeStruct((n, LANES), F32),
    )(partials, cnt_partials, h, g, b)


def _edge_update(hh, ht, e, wt, b, g, bb):
    m = e.shape[0]
    L = LANES

    def body(hh_ref, ht_ref, e_ref, w_ref, b_ref, g_ref, bb_ref, o_ref):
        ev = e_ref[...]
        w = w_ref[...]
        y = (_dot(hh_ref[...], w[0:L]) + _dot(ev, w[L:2 * L])
             + _dot(ht_ref[...], w[2 * L:3 * L]) + b_ref[...])
        o_ref[...] = _lnorm(_leaky(y) + ev, g_ref[...], bb_ref[...])

    edge = pl.BlockSpec((TE, LANES), lambda i: (i, 0))
    return pl.pallas_call(
        body,
        grid=(m // TE,),
        in_specs=[edge, edge, edge, _full((3 * LANES, LANES)),
                  _full((1, LANES)), _full((1, LANES)), _full((1, LANES))],
        out_specs=edge,
        out_shape=jax.ShapeDtypeStruct((m, LANES), F32),
    )(hh, ht, e, wt, b, g, bb)


def _classifier(e2, e0, hh, h0h, htl, h0t, w1t, b1, w2, b2):
    m = e2.shape[0]
    L = LANES

    def body(e2_ref, e0_ref, hh_ref, h0h_ref, ht_ref, h0t_ref,
             w_ref, b1_ref, w2_ref, b2_ref, o_ref):
        w = w_ref[...]
        y = (_dot(e2_ref[...], w[0:L]) + _dot(e0_ref[...], w[L:2 * L])
             + _dot(hh_ref[...], w[2 * L:3 * L])
             + _dot(h0h_ref[...], w[3 * L:4 * L])
             + _dot(ht_ref[...], w[4 * L:5 * L])
             + _dot(h0t_ref[...], w[5 * L:6 * L]) + b1_ref[...])
        o1 = _leaky(y)
        o_ref[...] = (jnp.sum(o1 * w2_ref[...], axis=1, keepdims=True)
                      + b2_ref[...])

    edge = pl.BlockSpec((TE, LANES), lambda i: (i, 0))
    return pl.pallas_call(
        body,
        grid=(m // TE,),
        in_specs=[edge, edge, edge, edge, edge, edge,
                  _full((6 * LANES, LANES)), _full((1, LANES)),
                  _full((1, LANES)), _full((1, 1))],
        out_specs=pl.BlockSpec((TE, 1), lambda i: (i, 0)),
        out_shape=jax.ShapeDtypeStruct((m, 1), F32),
    )(e2, e0, hh, h0h, htl, h0t, w1t, b1, w2, b2)


def _gather_many(table, idxs):
    n, d = table.shape
    r = idxs[0].shape[0] // 128
    b = -(-r // NW)
    nbuf = 2 if d > LANES else 4
    nio = len(idxs)

    def body(tab_hbm, *refs):
        idx_hbms = refs[:nio]
        out_hbms = refs[nio:2 * nio]
        idx_all = refs[2 * nio]
        rows = refs[2 * nio + 1:2 * nio + 1 + nbuf]
        sg = refs[2 * nio + 1 + nbuf:2 * nio + 1 + 2 * nbuf]
        so = refs[2 * nio + 1 + 2 * nbuf:2 * nio + 1 + 3 * nbuf]
        c = jax.lax.axis_index("c")
        s = jax.lax.axis_index("s")
        w = s * 2 + c
        start = w * b
        nc = jnp.clip(r - start, 0, b)
        cp = jnp.minimum(start, r - b)
        off = start - cp

        for idx_hbm, out_hbm in zip(idx_hbms, out_hbms):
            pltpu.sync_copy(idx_hbm.at[pl.ds(cp * 128, b * 128)], idx_all)

            def step(p, carry):
                for u in range(nbuf):
                    j = p * nbuf + u

                    @pl.when(j < nc)
                    def _(j=j, u=u):
                        @pl.when(p > 0)
                        def _():
                            pltpu.make_async_copy(
                                rows[u], out_hbm.at[pl.ds(0, 128)],
                                so[u]).wait()
                        o = pl.multiple_of((off + j) * 128, 128)
                        pltpu.async_copy(
                            tab_hbm.at[idx_all.at[pl.ds(o, 128)]],
                            rows[u], sg[u])
                for u in range(nbuf):
                    j = p * nbuf + u

                    @pl.when(j < nc)
                    def _(j=j, u=u):
                        pltpu.make_async_copy(
                            tab_hbm.at[idx_all.at[pl.ds(0, 128)]],
                            rows[u], sg[u]).wait()
                        pltpu.async_copy(
                            rows[u], out_hbm.at[pl.ds((start + j) * 128, 128)],
                            so[u])
                return carry

            jax.lax.fori_loop(0, -(-b // nbuf), step, 0)
            for u in range(nbuf):
                @pl.when(u < nc)
                def _(u=u):
                    pltpu.make_async_copy(
                        rows[u], out_hbm.at[pl.ds(0, 128)], so[u]).wait()

    f = pl.kernel(
        body,
        out_type=tuple(jax.ShapeDtypeStruct((r * 128, d), F32)
                       for _ in range(nio)),
        mesh=plsc.VectorSubcoreMesh(core_axis_name="c", subcore_axis_name="s"),
        scratch_types=(
            [pltpu.VMEM((b * 128,), jnp.int32)]
            + [pltpu.VMEM((128, d), F32) for _ in range(nbuf)]
            + [pltpu.SemaphoreType.DMA for _ in range(2 * nbuf)]
        ),
    )
    out = f(table, *idxs)
    if not isinstance(out, (list, tuple)):
        out = (out,)
    return tuple(out)


def _scatter_add(mf, mb, idxf2d, idxb2d, n):
    r = mf.shape[0] // 128
    b = idxf2d.shape[0] // 16
    nbuf = 2
    npad = ((n + 2047) // 2048) * 2048
    npt = npad // 16

    def body(mf_hbm, mb_hbm, if_hbm, ib_hbm, outp_hbm, outc_hbm,
             tab_sh, cnt_sh, idx_v, ones_v, cbuf_v, *rest):
        rows = rest[:nbuf]
        sm = rest[nbuf:2 * nbuf]
        ss = rest[2 * nbuf:3 * nbuf]
        c = jax.lax.axis_index("c")
        s = jax.lax.axis_index("s")
        zv = jnp.zeros((16,), F32)
        ov = jnp.ones((16,), F32)

        def zfill(i, carry):
            rows[0][i // 8, pl.ds((i % 8) * 16, 16)] = zv
            return carry
        jax.lax.fori_loop(0, 128 * 8, zfill, 0)

        def ofill(i, carry):
            ones_v[pl.ds(i * 16, 16)] = ov
            cbuf_v[pl.ds(i * 16, 16)] = zv
            return carry
        jax.lax.fori_loop(0, 8, ofill, 0)

        def ztab(z, carry):
            pltpu.sync_copy(rows[0], tab_sh.at[pl.ds(s * npt + z * 128, 128)])
            pltpu.sync_copy(cbuf_v, cnt_sh.at[pl.ds(s * npt + z * 128, 128)])
            return carry
        jax.lax.fori_loop(0, npt // 128, ztab, 0)
        plsc.subcore_barrier()

        start = s * b
        nc = jnp.clip(r - start, 0, b)

        for d_id, (msg_hbm, idx_hbm) in enumerate(
                ((mf_hbm, if_hbm), (mb_hbm, ib_hbm))):
            @pl.when(c == d_id)
            def _(msg_hbm=msg_hbm, idx_hbm=idx_hbm):
                pltpu.sync_copy(idx_hbm.at[pl.ds(start, b)], idx_v)

                def step(p, carry):
                    for u in range(nbuf):
                        j = p * nbuf + u

                        @pl.when(j < nc)
                        def _(j=j, u=u):
                            @pl.when(p > 0)
                            def _():
                                pltpu.make_async_copy(
                                    rows[u], tab_sh.at[pl.ds(0, 128)],
                                    ss[u]).wait()
                            pltpu.async_copy(
                                msg_hbm.at[pl.ds((start + j) * 128, 128)],
                                rows[u], sm[u])
                    for u in range(nbuf):
                        j = p * nbuf + u

                        @pl.when(j < nc)
                        def _(j=j, u=u):
                            pltpu.make_async_copy(
                                msg_hbm.at[pl.ds(0, 128)], rows[u],
                                sm[u]).wait()
                            pltpu.async_copy(rows[u], tab_sh.at[idx_v.at[j]],
                                             ss[u], add=True)
                            pltpu.sync_copy(ones_v, cnt_sh.at[idx_v.at[j]],
                                            add=True)
                    return carry

                jax.lax.fori_loop(0, -(-b // nbuf), step, 0)
                for u in range(nbuf):
                    @pl.when(u < nc)
                    def _(u=u):
                        pltpu.make_async_copy(
                            rows[u], tab_sh.at[pl.ds(0, 128)], ss[u]).wait()
        plsc.subcore_barrier()

        def wout(z, carry):
            a = s * npt + z * 128
            pltpu.sync_copy(tab_sh.at[pl.ds(a, 128)], rows[0])
            pltpu.sync_copy(rows[0], outp_hbm.at[c, pl.ds(a, 128)])
            pltpu.sync_copy(cnt_sh.at[pl.ds(a, 128)], cbuf_v)
            pltpu.sync_copy(cbuf_v, outc_hbm.at[c, pl.ds(a, 128)])
            return carry
        jax.lax.fori_loop(0, npt // 128, wout, 0)

    f = pl.kernel(
        body,
        out_type=(jax.ShapeDtypeStruct((2, npad, LANES), F32),
                  jax.ShapeDtypeStruct((2, npad), F32)),
        mesh=plsc.VectorSubcoreMesh(core_axis_name="c", subcore_axis_name="s"),
        scratch_types=(
            [pltpu.VMEM_SHARED((npad, LANES), F32),
             pltpu.VMEM_SHARED((npad,), F32),
             pltpu.VMEM((b, 128), jnp.int32),
             pltpu.VMEM((128,), F32),
             pltpu.VMEM((128,), F32)]
            + [pltpu.VMEM((128, LANES), F32) for _ in range(nbuf)]
            + [pltpu.SemaphoreType.DMA for _ in range(2 * nbuf)]
        ),
    )
    return f(mf, mb, idxf2d, idxb2d)


def kernel(ht, r_tensor, entity_feat, relation_feat, p_selections, queries,
           params):
    del p_selections
    m = ht.shape[0]
    n = entity_feat.shape[0]
    npad = ((n + 2047) // 2048) * 2048
    ef_pad = jnp.pad(entity_feat, ((0, npad - n), (0, 0)))

    hix = ht[:, 0].astype(jnp.int32)
    tix = ht[:, 1].astype(jnp.int32)
    rix = r_tensor.astype(jnp.int32)
    q2d = queries.reshape(m, 1)

    r_chunks = m // 128
    bslot = ((-(-r_chunks // 16) + 7) // 8) * 8
    rpad = 16 * bslot
    hix2d = jnp.pad(hix.reshape(r_chunks, 128),
                    ((0, rpad - r_chunks), (0, 0)))
    tix2d = jnp.pad(tix.reshape(r_chunks, 128),
                    ((0, rpad - r_chunks), (0, 0)))

    p = params
    row = lambda v: v.reshape(1, -1)

    h0 = _node_encoder(ef_pad, p['ent_in_W'].T, row(p['ent_in_b']),
                       row(p['ln_ent_g']), row(p['ln_ent_b']))
    t0 = _rel_table(relation_feat, p['edge_in_W'][:, :LANES].T,
                    p['rel_embed'])
    t0g, = _gather_many(t0, [rix])
    e0 = _edge_encoder(t0g, q2d, row(p['edge_in_W'][:, LANES]),
                       row(p['edge_in_b']), row(p['ln_edge_g']),
                       row(p['ln_edge_b']))

    h0h, h0t = _gather_many(h0, [hix, tix])

    h, e = h0, e0
    hh, htl = h0h, h0t
    for lp in p['layers']:
        mfv, mbv = _msg(hh, htl, e, t0g, lp['fwd_W'].T, lp['back_W'].T,
                        row(lp['fwd_b']), row(lp['back_b']))
        partials, cnt_partials = _scatter_add(mfv, mbv, tix2d, hix2d, npad)
        h = _h_update(partials, cnt_partials, h, row(lp['mp_ln_g']),
                      row(lp['mp_ln_b']))
        hh, htl = _gather_many(h, [hix, tix])
        e = _edge_update(hh, htl, e, lp['eu_W'].T, row(lp['eu_b']),
                         row(lp['eu_ln_g']), row(lp['eu_ln_b']))

    return _classifier(e, e0, hh, h0h, htl, h0t, p['cls_W1'].T,
                       row(p['cls_b1']), row(p['cls_W2']),
                       p['cls_b2'].reshape(1, 1))


# speedup vs baseline: 5.1756x; 1.0131x over previous
import functools

import jax
import jax.numpy as jnp
from jax.experimental import pallas as pl
from jax.experimental.pallas import tpu as pltpu
from jax.experimental.pallas import tpu_sc as plsc

F32 = jnp.float32
LANES = 128
NW = 32
TE = 2000
TN = 2048


def _leaky(x):
    return jnp.where(x >= 0, x, 0.01 * x)


def _lnorm(x, g, b):
    mu = x.mean(axis=-1, keepdims=True)
    var = ((x - mu) ** 2).mean(axis=-1, keepdims=True)
    return (x - mu) / jnp.sqrt(var + 1e-5) * g + b


def _dot(a, b):
    return jnp.dot(a, b, preferred_element_type=F32)


def _full(shape):
    return pl.BlockSpec(shape, lambda i: (0,) * len(shape))


def _node_encoder(x, wt, b, g, bb):
    n = x.shape[0]

    def body(x_ref, wt_ref, b_ref, g_ref, bb_ref, o_ref):
        y = _dot(x_ref[...], wt_ref[...]) + b_ref[...]
        o_ref[...] = _lnorm(_leaky(y), g_ref[...], bb_ref[...])

    return pl.pallas_call(
        body,
        grid=(n // TN,),
        in_specs=[
            pl.BlockSpec((TN, LANES), lambda i: (i, 0)),
            _full((LANES, LANES)), _full((1, LANES)), _full((1, LANES)),
            _full((1, LANES)),
        ],
        out_specs=pl.BlockSpec((TN, LANES), lambda i: (i, 0)),
        out_shape=jax.ShapeDtypeStruct((n, LANES), F32),
    )(x, wt, b, g, bb)


def _rel_table(relation_feat, w128t, rel_embed):
    nr = relation_feat.shape[0]

    def body(rf_ref, wt_ref, re_ref, o_ref):
        o_ref[:, 0:LANES] = _dot(rf_ref[...], wt_ref[...])
        o_ref[:, LANES:2 * LANES] = re_ref[...]

    return pl.pallas_call(
        body,
        grid=(1,),
        in_specs=[_full((nr, LANES)), _full((LANES, LANES)),
                  _full((nr, LANES))],
        out_specs=_full((nr, 2 * LANES)),
        out_shape=jax.ShapeDtypeStruct((nr, 2 * LANES), F32),
    )(relation_feat, w128t, rel_embed)


def _edge_encoder(t0g, q, wq, b, g, bb):
    m = t0g.shape[0]

    def body(r1_ref, q_ref, wq_ref, b_ref, g_ref, bb_ref, o_ref):
        y = r1_ref[...] + q_ref[...] * wq_ref[...] + b_ref[...]
        o_ref[...] = _lnorm(_leaky(y), g_ref[...], bb_ref[...])

    return pl.pallas_call(
        body,
        grid=(m // TE,),
        in_specs=[
            pl.BlockSpec((TE, LANES), lambda i: (i, 0)),
            pl.BlockSpec((TE, 1), lambda i: (i, 0)),
            _full((1, LANES)), _full((1, LANES)), _full((1, LANES)),
            _full((1, LANES)),
        ],
        out_specs=pl.BlockSpec((TE, LANES), lambda i: (i, 0)),
        out_shape=jax.ShapeDtypeStruct((m, LANES), F32),
    )(t0g, q, wq, b, g, bb)


def _msg(hh, ht, e, t0g, wft, wbt, bf, bb):
    m = e.shape[0]
    L = LANES

    def body(hh_ref, ht_ref, e_ref, r_ref, wf_ref, wb_ref, bf_ref, bb_ref,
             mf_ref, mb_ref):
        ev = e_ref[...]
        rv = r_ref[...]
        er = ev * rv
        wf = wf_ref[...]
        wb = wb_ref[...]
        hhv = hh_ref[...]
        htv = ht_ref[...]
        mf_ref[...] = (_dot(hhv, wf[0:L]) + _dot(ev, wf[L:2 * L])
                       + _dot(hhv * rv, wf[2 * L:3 * L])
                       + _dot(er, wf[3 * L:4 * L]) + bf_ref[...])
        mb_ref[...] = (_dot(htv, wb[0:L]) + _dot(ev, wb[L:2 * L])
                       + _dot(htv * rv, wb[2 * L:3 * L])
                       + _dot(er, wb[3 * L:4 * L]) + bb_ref[...])

    edge = pl.BlockSpec((TE, LANES), lambda i: (i, 0))
    return pl.pallas_call(
        body,
        grid=(m // TE,),
        in_specs=[
            edge, edge, edge,
            pl.BlockSpec((TE, LANES), lambda i: (i, 1)),
            _full((4 * LANES, LANES)), _full((4 * LANES, LANES)),
            _full((1, LANES)), _full((1, LANES)),
        ],
        out_specs=[edge, edge],
        out_shape=[jax.ShapeDtypeStruct((m, LANES), F32),
                   jax.ShapeDtypeStruct((m, LANES), F32)],
    )(hh, ht, e, t0g, wft, wbt, bf, bb)


def _h_update(partials, cnt_partials, h, g, b):
    n = h.shape[0]

    def body(p_ref, c_ref, h_ref, g_ref, b_ref, o_ref):
        agg = p_ref[0] + p_ref[1]
        cnt = (c_ref[0] + c_ref[1])[:, None]
        o_ref[...] = _lnorm(_leaky(agg / cnt) + h_ref[...],
                            g_ref[...], b_ref[...])

    return pl.pallas_call(
        body,
        grid=(n // TN,),
        in_specs=[
            pl.BlockSpec((2, TN, LANES), lambda i: (0, i, 0)),
            pl.BlockSpec((2, TN), lambda i: (0, i)),
            pl.BlockSpec((TN, LANES), lambda i: (i, 0)),
            _full((1, LANES)), _full((1, LANES)),
        ],
        out_specs=pl.BlockSpec((TN, LANES), lambda i: (i, 0)),
        out_shape=jax.ShapeDtypeStruct((n, LANES), F32),
    )(partials, cnt_partials, h, g, b)


def _edge_update(hh, ht, e, wt, b, g, bb):
    m = e.shape[0]
    L = LANES

    def body(hh_ref, ht_ref, e_ref, w_ref, b_ref, g_ref, bb_ref, o_ref):
        ev = e_ref[...]
        w = w_ref[...]
        y = (_dot(hh_ref[...], w[0:L]) + _dot(ev, w[L:2 * L])
             + _dot(ht_ref[...], w[2 * L:3 * L]) + b_ref[...])
        o_ref[...] = _lnorm(_leaky(y) + ev, g_ref[...], bb_ref[...])

    edge = pl.BlockSpec((TE, LANES), lambda i: (i, 0))
    return pl.pallas_call(
        body,
        grid=(m // TE,),
        in_specs=[edge, edge, edge, _full((3 * LANES, LANES)),
                  _full((1, LANES)), _full((1, LANES)), _full((1, LANES))],
        out_specs=edge,
        out_shape=jax.ShapeDtypeStruct((m, LANES), F32),
    )(hh, ht, e, wt, b, g, bb)


def _classifier(e2, e0, hh, h0h, htl, h0t, w1t, b1, w2, b2):
    m = e2.shape[0]
    L = LANES

    def body(e2_ref, e0_ref, hh_ref, h0h_ref, ht_ref, h0t_ref,
             w_ref, b1_ref, w2_ref, b2_ref, o_ref):
        w = w_ref[...]
        y = (_dot(e2_ref[...], w[0:L]) + _dot(e0_ref[...], w[L:2 * L])
             + _dot(hh_ref[...], w[2 * L:3 * L])
             + _dot(h0h_ref[...], w[3 * L:4 * L])
             + _dot(ht_ref[...], w[4 * L:5 * L])
             + _dot(h0t_ref[...], w[5 * L:6 * L]) + b1_ref[...])
        o1 = _leaky(y)
        o_ref[...] = (jnp.sum(o1 * w2_ref[...], axis=1, keepdims=True)
                      + b2_ref[...])

    edge = pl.BlockSpec((TE, LANES), lambda i: (i, 0))
    return pl.pallas_call(
        body,
        grid=(m // TE,),
        in_specs=[edge, edge, edge, edge, edge, edge,
                  _full((6 * LANES, LANES)), _full((1, LANES)),
                  _full((1, LANES)), _full((1, 1))],
        out_specs=pl.BlockSpec((TE, 1), lambda i: (i, 0)),
        out_shape=jax.ShapeDtypeStruct((m, 1), F32),
    )(e2, e0, hh, h0h, htl, h0t, w1t, b1, w2, b2)


def _gather_many(table, idxs):
    n, d = table.shape
    r = idxs[0].shape[0] // 128
    b = -(-r // NW)
    nbuf = 3 if d > LANES else 6
    nio = len(idxs)

    def body(tab_hbm, *refs):
        idx_hbms = refs[:nio]
        out_hbms = refs[nio:2 * nio]
        idx_all = refs[2 * nio]
        rows = refs[2 * nio + 1:2 * nio + 1 + nbuf]
        sg = refs[2 * nio + 1 + nbuf:2 * nio + 1 + 2 * nbuf]
        so = refs[2 * nio + 1 + 2 * nbuf:2 * nio + 1 + 3 * nbuf]
        c = jax.lax.axis_index("c")
        s = jax.lax.axis_index("s")
        w = s * 2 + c
        start = w * b
        nc = jnp.clip(r - start, 0, b)
        cp = jnp.minimum(start, r - b)
        off = start - cp

        for idx_hbm, out_hbm in zip(idx_hbms, out_hbms):
            pltpu.sync_copy(idx_hbm.at[pl.ds(cp * 128, b * 128)], idx_all)

            def step(p, carry):
                for u in range(nbuf):
                    j = p * nbuf + u

                    @pl.when(j < nc)
                    def _(j=j, u=u):
                        @pl.when(p > 0)
                        def _():
                            pltpu.make_async_copy(
                                rows[u], out_hbm.at[pl.ds(0, 128)],
                                so[u]).wait()
                        o = pl.multiple_of((off + j) * 128, 128)
                        pltpu.async_copy(
                            tab_hbm.at[idx_all.at[pl.ds(o, 128)]],
                            rows[u], sg[u])
                for u in range(nbuf):
                    j = p * nbuf + u

                    @pl.when(j < nc)
                    def _(j=j, u=u):
                        pltpu.make_async_copy(
                            tab_hbm.at[idx_all.at[pl.ds(0, 128)]],
                            rows[u], sg[u]).wait()
                        pltpu.async_copy(
                            rows[u], out_hbm.at[pl.ds((start + j) * 128, 128)],
                            so[u])
                return carry

            jax.lax.fori_loop(0, -(-b // nbuf), step, 0)
            for u in range(nbuf):
                @pl.when(u < nc)
                def _(u=u):
                    pltpu.make_async_copy(
                        rows[u], out_hbm.at[pl.ds(0, 128)], so[u]).wait()

    f = pl.kernel(
        body,
        out_type=tuple(jax.ShapeDtypeStruct((r * 128, d), F32)
                       for _ in range(nio)),
        mesh=plsc.VectorSubcoreMesh(core_axis_name="c", subcore_axis_name="s"),
        scratch_types=(
            [pltpu.VMEM((b * 128,), jnp.int32)]
            + [pltpu.VMEM((128, d), F32) for _ in range(nbuf)]
            + [pltpu.SemaphoreType.DMA for _ in range(2 * nbuf)]
        ),
    )
    out = f(table, *idxs)
    if not isinstance(out, (list, tuple)):
        out = (out,)
    return tuple(out)


def _scatter_add(mf, mb, idxf2d, idxb2d, n, with_counts):
    r = mf.shape[0] // 128
    b = idxf2d.shape[0] // 16
    nbuf = 2
    npad = ((n + 2047) // 2048) * 2048
    npt = npad // 16

    def body(mf_hbm, mb_hbm, if_hbm, ib_hbm, outp_hbm, outc_hbm,
             tab_sh, cnt_sh, idx_v, ones_v, cbuf_v, *rest):
        rows = rest[:nbuf]
        sm = rest[nbuf:2 * nbuf]
        ss = rest[2 * nbuf:3 * nbuf]
        sc = rest[3 * nbuf]
        c = jax.lax.axis_index("c")
        s = jax.lax.axis_index("s")
        zv = jnp.zeros((16,), F32)
        ov = jnp.ones((16,), F32)

        def zfill(i, carry):
            rows[0][i // 8, pl.ds((i % 8) * 16, 16)] = zv
            return carry
        jax.lax.fori_loop(0, 128 * 8, zfill, 0)

        def ofill(i, carry):
            ones_v[pl.ds(i * 16, 16)] = ov
            cbuf_v[pl.ds(i * 16, 16)] = zv
            return carry
        jax.lax.fori_loop(0, 8, ofill, 0)

        def ztab(z, carry):
            pltpu.sync_copy(rows[0], tab_sh.at[pl.ds(s * npt + z * 128, 128)])
            if with_counts:
                pltpu.sync_copy(cbuf_v,
                                cnt_sh.at[pl.ds(s * npt + z * 128, 128)])
            return carry
        jax.lax.fori_loop(0, npt // 128, ztab, 0)
        plsc.subcore_barrier()

        start = s * b
        nc = jnp.clip(r - start, 0, b)

        for d_id, (msg_hbm, idx_hbm) in enumerate(
                ((mf_hbm, if_hbm), (mb_hbm, ib_hbm))):
            @pl.when(c == d_id)
            def _(msg_hbm=msg_hbm, idx_hbm=idx_hbm):
                pltpu.sync_copy(idx_hbm.at[pl.ds(start, b)], idx_v)

                def step(p, carry):
                    for u in range(nbuf):
                        j = p * nbuf + u

                        @pl.when(j < nc)
                        def _(j=j, u=u):
                            @pl.when(p > 0)
                            def _():
                                pltpu.make_async_copy(
                                    rows[u], tab_sh.at[pl.ds(0, 128)],
                                    ss[u]).wait()
                            pltpu.async_copy(
                                msg_hbm.at[pl.ds((start + j) * 128, 128)],
                                rows[u], sm[u])
                    for u in range(nbuf):
                        j = p * nbuf + u

                        @pl.when(j < nc)
                        def _(j=j, u=u):
                            pltpu.make_async_copy(
                                msg_hbm.at[pl.ds(0, 128)], rows[u],
                                sm[u]).wait()
                            pltpu.async_copy(rows[u], tab_sh.at[idx_v.at[j]],
                                             ss[u], add=True)
                            if with_counts:
                                pltpu.async_copy(ones_v,
                                                 cnt_sh.at[idx_v.at[j]], sc,
                                                 add=True)
                    return carry

                jax.lax.fori_loop(0, -(-b // nbuf), step, 0)
                for u in range(nbuf):
                    @pl.when(u < nc)
                    def _(u=u):
                        pltpu.make_async_copy(
                            rows[u], tab_sh.at[pl.ds(0, 128)], ss[u]).wait()
                if with_counts:
                    def cdrain(j, carry):
                        @pl.when(j < nc)
                        def _():
                            pltpu.make_async_copy(
                                ones_v, cnt_sh.at[pl.ds(0, 128)], sc).wait()
                        return carry
                    jax.lax.fori_loop(0, b, cdrain, 0)
        plsc.subcore_barrier()

        def wout(z, carry):
            a = s * npt + z * 128
            pltpu.sync_copy(tab_sh.at[pl.ds(a, 128)], rows[0])
            pltpu.sync_copy(rows[0], outp_hbm.at[c, pl.ds(a, 128)])
            if with_counts:
                pltpu.sync_copy(cnt_sh.at[pl.ds(a, 128)], cbuf_v)
                pltpu.sync_copy(cbuf_v, outc_hbm.at[c, pl.ds(a, 128)])
            return carry
        jax.lax.fori_loop(0, npt // 128, wout, 0)

    f = pl.kernel(
        body,
        out_type=(jax.ShapeDtypeStruct((2, npad, LANES), F32),
                  jax.ShapeDtypeStruct((2, npad), F32)),
        mesh=plsc.VectorSubcoreMesh(core_axis_name="c", subcore_axis_name="s"),
        scratch_types=(
            [pltpu.VMEM_SHARED((npad, LANES), F32),
             pltpu.VMEM_SHARED((npad,), F32),
             pltpu.VMEM((b, 128), jnp.int32),
             pltpu.VMEM((128,), F32),
             pltpu.VMEM((128,), F32)]
            + [pltpu.VMEM((128, LANES), F32) for _ in range(nbuf)]
            + [pltpu.SemaphoreType.DMA for _ in range(2 * nbuf + 1)]
        ),
    )
    return f(mf, mb, idxf2d, idxb2d)


def kernel(ht, r_tensor, entity_feat, relation_feat, p_selections, queries,
           params):
    del p_selections
    m = ht.shape[0]
    n = entity_feat.shape[0]
    npad = ((n + 2047) // 2048) * 2048
    ef_pad = jnp.pad(entity_feat, ((0, npad - n), (0, 0)))

    hix = ht[:, 0].astype(jnp.int32)
    tix = ht[:, 1].astype(jnp.int32)
    rix = r_tensor.astype(jnp.int32)
    q2d = queries.reshape(m, 1)

    r_chunks = m // 128
    bslot = ((-(-r_chunks // 16) + 7) // 8) * 8
    rpad = 16 * bslot
    hix2d = jnp.pad(hix.reshape(r_chunks, 128),
                    ((0, rpad - r_chunks), (0, 0)))
    tix2d = jnp.pad(tix.reshape(r_chunks, 128),
                    ((0, rpad - r_chunks), (0, 0)))

    p = params
    row = lambda v: v.reshape(1, -1)

    h0 = _node_encoder(ef_pad, p['ent_in_W'].T, row(p['ent_in_b']),
                       row(p['ln_ent_g']), row(p['ln_ent_b']))
    t0 = _rel_table(relation_feat, p['edge_in_W'][:, :LANES].T,
                    p['rel_embed'])
    t0g, = _gather_many(t0, [rix])
    e0 = _edge_encoder(t0g, q2d, row(p['edge_in_W'][:, LANES]),
                       row(p['edge_in_b']), row(p['ln_edge_g']),
                       row(p['ln_edge_b']))

    h0h, h0t = _gather_many(h0, [hix, tix])

    h, e = h0, e0
    hh, htl = h0h, h0t
    cnt_partials = None
    for li, lp in enumerate(p['layers']):
        mfv, mbv = _msg(hh, htl, e, t0g, lp['fwd_W'].T, lp['back_W'].T,
                        row(lp['fwd_b']), row(lp['back_b']))
        partials, cnts = _scatter_add(mfv, mbv, tix2d, hix2d, npad,
                                      with_counts=(li == 0))
        if li == 0:
            cnt_partials = cnts
        h = _h_update(partials, cnt_partials, h, row(lp['mp_ln_g']),
                      row(lp['mp_ln_b']))
        hh, htl = _gather_many(h, [hix, tix])
        e = _edge_update(hh, htl, e, lp['eu_W'].T, row(lp['eu_b']),
                         row(lp['eu_ln_g']), row(lp['eu_ln_b']))

    return _classifier(e, e0, hh, h0h, htl, h0t, p['cls_W1'].T,
                       row(p['cls_b1']), row(p['cls_W2']),
                       p['cls_b2'].reshape(1, 1))


# speedup vs baseline: 5.8191x; 1.1243x over previous
import functools

import jax
import jax.numpy as jnp
from jax.experimental import pallas as pl
from jax.experimental.pallas import tpu as pltpu
from jax.experimental.pallas import tpu_sc as plsc

F32 = jnp.float32
LANES = 128
NW = 32
TE = 2000
TN = 2048


def _leaky(x):
    return jnp.where(x >= 0, x, 0.01 * x)


def _lnorm(x, g, b):
    mu = x.mean(axis=-1, keepdims=True)
    var = ((x - mu) ** 2).mean(axis=-1, keepdims=True)
    return (x - mu) / jnp.sqrt(var + 1e-5) * g + b


def _dot(a, b):
    return jnp.dot(a, b, preferred_element_type=F32)


def _full(shape):
    return pl.BlockSpec(shape, lambda i: (0,) * len(shape))


def _node_encoder(x, wt, b, g, bb):
    n = x.shape[0]

    def body(x_ref, wt_ref, b_ref, g_ref, bb_ref, o_ref):
        y = _dot(x_ref[...], wt_ref[...]) + b_ref[...]
        o_ref[...] = _lnorm(_leaky(y), g_ref[...], bb_ref[...])

    return pl.pallas_call(
        body,
        grid=(n // TN,),
        in_specs=[
            pl.BlockSpec((TN, LANES), lambda i: (i, 0)),
            _full((LANES, LANES)), _full((1, LANES)), _full((1, LANES)),
            _full((1, LANES)),
        ],
        out_specs=pl.BlockSpec((TN, LANES), lambda i: (i, 0)),
        out_shape=jax.ShapeDtypeStruct((n, LANES), F32),
    )(x, wt, b, g, bb)


def _rel_table(relation_feat, w128t, rel_embed):
    nr = relation_feat.shape[0]

    def body(rf_ref, wt_ref, re_ref, o_ref):
        o_ref[:, 0:LANES] = _dot(rf_ref[...], wt_ref[...])
        o_ref[:, LANES:2 * LANES] = re_ref[...]

    return pl.pallas_call(
        body,
        grid=(1,),
        in_specs=[_full((nr, LANES)), _full((LANES, LANES)),
                  _full((nr, LANES))],
        out_specs=_full((nr, 2 * LANES)),
        out_shape=jax.ShapeDtypeStruct((nr, 2 * LANES), F32),
    )(relation_feat, w128t, rel_embed)


def _edge_encoder(t0g, q, wq, b, g, bb):
    m = t0g.shape[0]

    def body(r1_ref, q_ref, wq_ref, b_ref, g_ref, bb_ref, o_ref):
        y = r1_ref[...] + q_ref[...] * wq_ref[...] + b_ref[...]
        o_ref[...] = _lnorm(_leaky(y), g_ref[...], bb_ref[...])

    return pl.pallas_call(
        body,
        grid=(m // TE,),
        in_specs=[
            pl.BlockSpec((TE, LANES), lambda i: (i, 0)),
            pl.BlockSpec((TE, 1), lambda i: (i, 0)),
            _full((1, LANES)), _full((1, LANES)), _full((1, LANES)),
            _full((1, LANES)),
        ],
        out_specs=pl.BlockSpec((TE, LANES), lambda i: (i, 0)),
        out_shape=jax.ShapeDtypeStruct((m, LANES), F32),
    )(t0g, q, wq, b, g, bb)


def _msg(hh, ht, e, t0g, wft, wbt, bf, bb):
    m = e.shape[0]
    L = LANES

    def body(hh_ref, ht_ref, e_ref, r_ref, wf_ref, wb_ref, bf_ref, bb_ref,
             mf_ref, mb_ref):
        ev = e_ref[...]
        rv = r_ref[...]
        er = ev * rv
        wf = wf_ref[...]
        wb = wb_ref[...]
        hhv = hh_ref[...]
        htv = ht_ref[...]
        mf_ref[...] = (_dot(hhv, wf[0:L]) + _dot(ev, wf[L:2 * L])
                       + _dot(hhv * rv, wf[2 * L:3 * L])
                       + _dot(er, wf[3 * L:4 * L]) + bf_ref[...])
        mb_ref[...] = (_dot(htv, wb[0:L]) + _dot(ev, wb[L:2 * L])
                       + _dot(htv * rv, wb[2 * L:3 * L])
                       + _dot(er, wb[3 * L:4 * L]) + bb_ref[...])

    edge = pl.BlockSpec((TE, LANES), lambda i: (i, 0))
    return pl.pallas_call(
        body,
        grid=(m // TE,),
        in_specs=[
            edge, edge, edge,
            pl.BlockSpec((TE, LANES), lambda i: (i, 1)),
            _full((4 * LANES, LANES)), _full((4 * LANES, LANES)),
            _full((1, LANES)), _full((1, LANES)),
        ],
        out_specs=[edge, edge],
        out_shape=[jax.ShapeDtypeStruct((m, LANES), F32),
                   jax.ShapeDtypeStruct((m, LANES), F32)],
    )(hh, ht, e, t0g, wft, wbt, bf, bb)


def _msg_terms(hv, tv, ev, rv, wf, wb, bf, bb):
    L = LANES
    er = ev * rv
    mf = (_dot(hv, wf[0:L]) + _dot(ev, wf[L:2 * L])
          + _dot(hv * rv, wf[2 * L:3 * L]) + _dot(er, wf[3 * L:4 * L]) + bf)
    mb = (_dot(tv, wb[0:L]) + _dot(ev, wb[L:2 * L])
          + _dot(tv * rv, wb[2 * L:3 * L]) + _dot(er, wb[3 * L:4 * L]) + bb)
    return mf, mb


def _enc_msg(t0g, q, wq, be, ge, bbe, hh, ht, wft, wbt, bf, bb):
    m = hh.shape[0]

    def body(t_ref, q_ref, wq_ref, be_ref, ge_ref, bbe_ref, hh_ref, ht_ref,
             wf_ref, wb_ref, bf_ref, bb_ref, e0_ref, mf_ref, mb_ref):
        t = t_ref[...]
        e0 = _lnorm(_leaky(t[:, 0:LANES] + q_ref[...] * wq_ref[...]
                           + be_ref[...]), ge_ref[...], bbe_ref[...])
        e0_ref[...] = e0
        mf, mb = _msg_terms(hh_ref[...], ht_ref[...], e0, t[:, LANES:],
                            wf_ref[...], wb_ref[...], bf_ref[...],
                            bb_ref[...])
        mf_ref[...] = mf
        mb_ref[...] = mb

    edge = pl.BlockSpec((TE, LANES), lambda i: (i, 0))
    return pl.pallas_call(
        body,
        grid=(m // TE,),
        in_specs=[
            pl.BlockSpec((TE, 2 * LANES), lambda i: (i, 0)),
            pl.BlockSpec((TE, 1), lambda i: (i, 0)),
            _full((1, LANES)), _full((1, LANES)), _full((1, LANES)),
            _full((1, LANES)), edge, edge,
            _full((4 * LANES, LANES)), _full((4 * LANES, LANES)),
            _full((1, LANES)), _full((1, LANES)),
        ],
        out_specs=[edge, edge, edge],
        out_shape=[jax.ShapeDtypeStruct((m, LANES), F32)] * 3,
    )(t0g, q, wq, be, ge, bbe, hh, ht, wft, wbt, bf, bb)


def _eu_msg(hh, ht, e, t0g, euwt, eub, eug, eubb, wft, wbt, bf, bb):
    m = e.shape[0]
    L = LANES

    def body(hh_ref, ht_ref, e_ref, r_ref, w_ref, b_ref, g_ref, bb_ref,
             wf_ref, wb_ref, bf_ref, bb2_ref, en_ref, mf_ref, mb_ref):
        ev = e_ref[...]
        hhv = hh_ref[...]
        htv = ht_ref[...]
        w = w_ref[...]
        y = (_dot(hhv, w[0:L]) + _dot(ev, w[L:2 * L])
             + _dot(htv, w[2 * L:3 * L]) + b_ref[...])
        en = _lnorm(_leaky(y) + ev, g_ref[...], bb_ref[...])
        en_ref[...] = en
        mf, mb = _msg_terms(hhv, htv, en, r_ref[...], wf_ref[...],
                            wb_ref[...], bf_ref[...], bb2_ref[...])
        mf_ref[...] = mf
        mb_ref[...] = mb

    edge = pl.BlockSpec((TE, LANES), lambda i: (i, 0))
    return pl.pallas_call(
        body,
        grid=(m // TE,),
        in_specs=[edge, edge, edge,
                  pl.BlockSpec((TE, LANES), lambda i: (i, 1)),
                  _full((3 * LANES, LANES)), _full((1, LANES)),
                  _full((1, LANES)), _full((1, LANES)),
                  _full((4 * LANES, LANES)), _full((4 * LANES, LANES)),
                  _full((1, LANES)), _full((1, LANES))],
        out_specs=[edge, edge, edge],
        out_shape=[jax.ShapeDtypeStruct((m, LANES), F32)] * 3,
    )(hh, ht, e, t0g, euwt, eub, eug, eubb, wft, wbt, bf, bb)


def _eu_cls(hh, ht, e, e0, h0h, h0t, euwt, eub, eug, eubb, w1t, b1, w2, b2):
    m = e.shape[0]
    L = LANES

    def body(hh_ref, ht_ref, e_ref, e0_ref, h0h_ref, h0t_ref, w_ref, b_ref,
             g_ref, bb_ref, w1_ref, b1_ref, w2_ref, b2_ref, o_ref):
        ev = e_ref[...]
        hhv = hh_ref[...]
        htv = ht_ref[...]
        w = w_ref[...]
        y = (_dot(hhv, w[0:L]) + _dot(ev, w[L:2 * L])
             + _dot(htv, w[2 * L:3 * L]) + b_ref[...])
        e2 = _lnorm(_leaky(y) + ev, g_ref[...], bb_ref[...])
        w1 = w1_ref[...]
        y1 = (_dot(e2, w1[0:L]) + _dot(e0_ref[...], w1[L:2 * L])
              + _dot(hhv, w1[2 * L:3 * L])
              + _dot(h0h_ref[...], w1[3 * L:4 * L])
              + _dot(htv, w1[4 * L:5 * L])
              + _dot(h0t_ref[...], w1[5 * L:6 * L]) + b1_ref[...])
        o1 = _leaky(y1)
        o_ref[...] = (jnp.sum(o1 * w2_ref[...], axis=1, keepdims=True)
                      + b2_ref[...])

    edge = pl.BlockSpec((TE, LANES), lambda i: (i, 0))
    return pl.pallas_call(
        body,
        grid=(m // TE,),
        in_specs=[edge, edge, edge, edge, edge, edge,
                  _full((3 * LANES, LANES)), _full((1, LANES)),
                  _full((1, LANES)), _full((1, LANES)),
                  _full((6 * LANES, LANES)), _full((1, LANES)),
                  _full((1, LANES)), _full((1, 1))],
        out_specs=pl.BlockSpec((TE, 1), lambda i: (i, 0)),
        out_shape=jax.ShapeDtypeStruct((m, 1), F32),
    )(hh, ht, e, e0, h0h, h0t, euwt, eub, eug, eubb, w1t, b1, w2, b2)


def _h_update(partials, cnt_partials, h, g, b):
    n = h.shape[0]

    def body(p_ref, c_ref, h_ref, g_ref, b_ref, o_ref):
        agg = p_ref[0] + p_ref[1]
        cnt = (c_ref[0] + c_ref[1])[:, None]
        o_ref[...] = _lnorm(_leaky(agg / cnt) + h_ref[...],
                            g_ref[...], b_ref[...])

    return pl.pallas_call(
        body,
        grid=(n // TN,),
        in_specs=[
            pl.BlockSpec((2, TN, LANES), lambda i: (0, i, 0)),
            pl.BlockSpec((2, TN), lambda i: (0, i)),
            pl.BlockSpec((TN, LANES), lambda i: (i, 0)),
            _full((1, LANES)), _full((1, LANES)),
        ],
        out_specs=pl.BlockSpec((TN, LANES), lambda i: (i, 0)),
        out_shape=jax.ShapeDtypeStruct((n, LANES), F32),
    )(partials, cnt_partials, h, g, b)


def _edge_update(hh, ht, e, wt, b, g, bb):
    m = e.shape[0]
    L = LANES

    def body(hh_ref, ht_ref, e_ref, w_ref, b_ref, g_ref, bb_ref, o_ref):
        ev = e_ref[...]
        w = w_ref[...]
        y = (_dot(hh_ref[...], w[0:L]) + _dot(ev, w[L:2 * L])
             + _dot(ht_ref[...], w[2 * L:3 * L]) + b_ref[...])
        o_ref[...] = _lnorm(_leaky(y) + ev, g_ref[...], bb_ref[...])

    edge = pl.BlockSpec((TE, LANES), lambda i: (i, 0))
    return pl.pallas_call(
        body,
        grid=(m // TE,),
        in_specs=[edge, edge, edge, _full((3 * LANES, LANES)),
                  _full((1, LANES)), _full((1, LANES)), _full((1, LANES))],
        out_specs=edge,
        out_shape=jax.ShapeDtypeStruct((m, LANES), F32),
    )(hh, ht, e, wt, b, g, bb)


def _classifier(e2, e0, hh, h0h, htl, h0t, w1t, b1, w2, b2):
    m = e2.shape[0]
    L = LANES

    def body(e2_ref, e0_ref, hh_ref, h0h_ref, ht_ref, h0t_ref,
             w_ref, b1_ref, w2_ref, b2_ref, o_ref):
        w = w_ref[...]
        y = (_dot(e2_ref[...], w[0:L]) + _dot(e0_ref[...], w[L:2 * L])
             + _dot(hh_ref[...], w[2 * L:3 * L])
             + _dot(h0h_ref[...], w[3 * L:4 * L])
             + _dot(ht_ref[...], w[4 * L:5 * L])
             + _dot(h0t_ref[...], w[5 * L:6 * L]) + b1_ref[...])
        o1 = _leaky(y)
        o_ref[...] = (jnp.sum(o1 * w2_ref[...], axis=1, keepdims=True)
                      + b2_ref[...])

    edge = pl.BlockSpec((TE, LANES), lambda i: (i, 0))
    return pl.pallas_call(
        body,
        grid=(m // TE,),
        in_specs=[edge, edge, edge, edge, edge, edge,
                  _full((6 * LANES, LANES)), _full((1, LANES)),
                  _full((1, LANES)), _full((1, 1))],
        out_specs=pl.BlockSpec((TE, 1), lambda i: (i, 0)),
        out_shape=jax.ShapeDtypeStruct((m, 1), F32),
    )(e2, e0, hh, h0h, htl, h0t, w1t, b1, w2, b2)


def _gather_many(table, idxs):
    n, d = table.shape
    r = idxs[0].shape[0] // 128
    b = -(-r // NW)
    nbuf = 3 if d > LANES else 6
    nio = len(idxs)

    def body(tab_hbm, *refs):
        idx_hbms = refs[:nio]
        out_hbms = refs[nio:2 * nio]
        idx_all = refs[2 * nio]
        rows = refs[2 * nio + 1:2 * nio + 1 + nbuf]
        sg = refs[2 * nio + 1 + nbuf:2 * nio + 1 + 2 * nbuf]
        so = refs[2 * nio + 1 + 2 * nbuf:2 * nio + 1 + 3 * nbuf]
        c = jax.lax.axis_index("c")
        s = jax.lax.axis_index("s")
        w = s * 2 + c
        start = w * b
        nc = jnp.clip(r - start, 0, b)
        cp = jnp.minimum(start, r - b)
        off = start - cp

        for idx_hbm, out_hbm in zip(idx_hbms, out_hbms):
            pltpu.sync_copy(idx_hbm.at[pl.ds(cp * 128, b * 128)], idx_all)

            def step(p, carry):
                for u in range(nbuf):
                    j = p * nbuf + u

                    @pl.when(j < nc)
                    def _(j=j, u=u):
                        @pl.when(p > 0)
                        def _():
                            pltpu.make_async_copy(
                                rows[u], out_hbm.at[pl.ds(0, 128)],
                                so[u]).wait()
                        o = pl.multiple_of((off + j) * 128, 128)
                        pltpu.async_copy(
                            tab_hbm.at[idx_all.at[pl.ds(o, 128)]],
                            rows[u], sg[u])
                for u in range(nbuf):
                    j = p * nbuf + u

                    @pl.when(j < nc)
                    def _(j=j, u=u):
                        pltpu.make_async_copy(
                            tab_hbm.at[idx_all.at[pl.ds(0, 128)]],
                            rows[u], sg[u]).wait()
                        pltpu.async_copy(
                            rows[u], out_hbm.at[pl.ds((start + j) * 128, 128)],
                            so[u])
                return carry

            jax.lax.fori_loop(0, -(-b // nbuf), step, 0)
            for u in range(nbuf):
                @pl.when(u < nc)
                def _(u=u):
                    pltpu.make_async_copy(
                        rows[u], out_hbm.at[pl.ds(0, 128)], so[u]).wait()

    f = pl.kernel(
        body,
        out_type=tuple(jax.ShapeDtypeStruct((r * 128, d), F32)
                       for _ in range(nio)),
        mesh=plsc.VectorSubcoreMesh(core_axis_name="c", subcore_axis_name="s"),
        scratch_types=(
            [pltpu.VMEM((b * 128,), jnp.int32)]
            + [pltpu.VMEM((128, d), F32) for _ in range(nbuf)]
            + [pltpu.SemaphoreType.DMA for _ in range(2 * nbuf)]
        ),
    )
    out = f(table, *idxs)
    if not isinstance(out, (list, tuple)):
        out = (out,)
    return tuple(out)


def _scatter_add(mf, mb, idxf2d, idxb2d, n, with_counts):
    r = mf.shape[0] // 128
    b = idxf2d.shape[0] // 16
    nbuf = 2
    npad = ((n + 2047) // 2048) * 2048
    npt = npad // 16

    def body(mf_hbm, mb_hbm, if_hbm, ib_hbm, outp_hbm, outc_hbm,
             tab_sh, cnt_sh, idx_v, ones_v, cbuf_v, *rest):
        rows = rest[:nbuf]
        sm = rest[nbuf:2 * nbuf]
        ss = rest[2 * nbuf:3 * nbuf]
        sc = rest[3 * nbuf]
        c = jax.lax.axis_index("c")
        s = jax.lax.axis_index("s")
        zv = jnp.zeros((16,), F32)
        ov = jnp.ones((16,), F32)

        def zfill(i, carry):
            rows[0][i // 8, pl.ds((i % 8) * 16, 16)] = zv
            return carry
        jax.lax.fori_loop(0, 128 * 8, zfill, 0)

        def ofill(i, carry):
            ones_v[pl.ds(i * 16, 16)] = ov
            cbuf_v[pl.ds(i * 16, 16)] = zv
            return carry
        jax.lax.fori_loop(0, 8, ofill, 0)

        def ztab(z, carry):
            pltpu.sync_copy(rows[0], tab_sh.at[pl.ds(s * npt + z * 128, 128)])
            if with_counts:
                pltpu.sync_copy(cbuf_v,
                                cnt_sh.at[pl.ds(s * npt + z * 128, 128)])
            return carry
        jax.lax.fori_loop(0, npt // 128, ztab, 0)
        plsc.subcore_barrier()

        start = s * b
        nc = jnp.clip(r - start, 0, b)

        for d_id, (msg_hbm, idx_hbm) in enumerate(
                ((mf_hbm, if_hbm), (mb_hbm, ib_hbm))):
            @pl.when(c == d_id)
            def _(msg_hbm=msg_hbm, idx_hbm=idx_hbm):
                pltpu.sync_copy(idx_hbm.at[pl.ds(start, b)], idx_v)

                def step(p, carry):
                    for u in range(nbuf):
                        j = p * nbuf + u

                        @pl.when(j < nc)
                        def _(j=j, u=u):
                            @pl.when(p > 0)
                            def _():
                                pltpu.make_async_copy(
                                    rows[u], tab_sh.at[pl.ds(0, 128)],
                                    ss[u]).wait()
                            pltpu.async_copy(
                                msg_hbm.at[pl.ds((start + j) * 128, 128)],
                                rows[u], sm[u])
                    for u in range(nbuf):
                        j = p * nbuf + u

                        @pl.when(j < nc)
                        def _(j=j, u=u):
                            pltpu.make_async_copy(
                                msg_hbm.at[pl.ds(0, 128)], rows[u],
                                sm[u]).wait()
                            pltpu.async_copy(rows[u], tab_sh.at[idx_v.at[j]],
                                             ss[u], add=True)
                            if with_counts:
                                pltpu.async_copy(ones_v,
                                                 cnt_sh.at[idx_v.at[j]], sc,
                                                 add=True)
                    return carry

                jax.lax.fori_loop(0, -(-b // nbuf), step, 0)
                for u in range(nbuf):
                    @pl.when(u < nc)
                    def _(u=u):
                        pltpu.make_async_copy(
                            rows[u], tab_sh.at[pl.ds(0, 128)], ss[u]).wait()
                if with_counts:
                    def cdrain(j, carry):
                        @pl.when(j < nc)
                        def _():
                            pltpu.make_async_copy(
                                ones_v, cnt_sh.at[pl.ds(0, 128)], sc).wait()
                        return carry
                    jax.lax.fori_loop(0, b, cdrain, 0)
        plsc.subcore_barrier()

        def wout(z, carry):
            a = s * npt + z * 128
            pltpu.sync_copy(tab_sh.at[pl.ds(a, 128)], rows[0])
            pltpu.sync_copy(rows[0], outp_hbm.at[c, pl.ds(a, 128)])
            if with_counts:
                pltpu.sync_copy(cnt_sh.at[pl.ds(a, 128)], cbuf_v)
                pltpu.sync_copy(cbuf_v, outc_hbm.at[c, pl.ds(a, 128)])
            return carry
        jax.lax.fori_loop(0, npt // 128, wout, 0)

    f = pl.kernel(
        body,
        out_type=(jax.ShapeDtypeStruct((2, npad, LANES), F32),
                  jax.ShapeDtypeStruct((2, npad), F32)),
        mesh=plsc.VectorSubcoreMesh(core_axis_name="c", subcore_axis_name="s"),
        scratch_types=(
            [pltpu.VMEM_SHARED((npad, LANES), F32),
             pltpu.VMEM_SHARED((npad,), F32),
             pltpu.VMEM((b, 128), jnp.int32),
             pltpu.VMEM((128,), F32),
             pltpu.VMEM((128,), F32)]
            + [pltpu.VMEM((128, LANES), F32) for _ in range(nbuf)]
            + [pltpu.SemaphoreType.DMA for _ in range(2 * nbuf + 1)]
        ),
    )
    return f(mf, mb, idxf2d, idxb2d)


def kernel(ht, r_tensor, entity_feat, relation_feat, p_selections, queries,
           params):
    del p_selections
    m = ht.shape[0]
    n = entity_feat.shape[0]
    npad = ((n + 2047) // 2048) * 2048
    ef_pad = jnp.pad(entity_feat, ((0, npad - n), (0, 0)))

    hix = ht[:, 0].astype(jnp.int32)
    tix = ht[:, 1].astype(jnp.int32)
    rix = r_tensor.astype(jnp.int32)
    q2d = queries.reshape(m, 1)

    r_chunks = m // 128
    bslot = ((-(-r_chunks // 16) + 7) // 8) * 8
    rpad = 16 * bslot
    hix2d = jnp.pad(hix.reshape(r_chunks, 128),
                    ((0, rpad - r_chunks), (0, 0)))
    tix2d = jnp.pad(tix.reshape(r_chunks, 128),
                    ((0, rpad - r_chunks), (0, 0)))

    p = params
    row = lambda v: v.reshape(1, -1)

    h0 = _node_encoder(ef_pad, p['ent_in_W'].T, row(p['ent_in_b']),
                       row(p['ln_ent_g']), row(p['ln_ent_b']))
    t0 = _rel_table(relation_feat, p['edge_in_W'][:, :LANES].T,
                    p['rel_embed'])
    t0g, = _gather_many(t0, [rix])
    h0h, h0t = _gather_many(h0, [hix, tix])

    lp1, lp2 = p['layers']
    e0, mf1, mb1 = _enc_msg(t0g, q2d, row(p['edge_in_W'][:, LANES]),
                            row(p['edge_in_b']), row(p['ln_edge_g']),
                            row(p['ln_edge_b']), h0h, h0t,
                            lp1['fwd_W'].T, lp1['back_W'].T,
                            row(lp1['fwd_b']), row(lp1['back_b']))
    partials, cnt_partials = _scatter_add(mf1, mb1, tix2d, hix2d, npad,
                                          with_counts=True)
    h1 = _h_update(partials, cnt_partials, h0, row(lp1['mp_ln_g']),
                   row(lp1['mp_ln_b']))
    h1h, h1t = _gather_many(h1, [hix, tix])

    e1, mf2, mb2 = _eu_msg(h1h, h1t, e0, t0g, lp1['eu_W'].T,
                           row(lp1['eu_b']), row(lp1['eu_ln_g']),
                           row(lp1['eu_ln_b']),
                           lp2['fwd_W'].T, lp2['back_W'].T,
                           row(lp2['fwd_b']), row(lp2['back_b']))
    partials2, _ = _scatter_add(mf2, mb2, tix2d, hix2d, npad,
                                with_counts=False)
    h2 = _h_update(partials2, cnt_partials, h1, row(lp2['mp_ln_g']),
                   row(lp2['mp_ln_b']))
    h2h, h2t = _gather_many(h2, [hix, tix])

    return _eu_cls(h2h, h2t, e1, e0, h0h, h0t, lp2['eu_W'].T,
                   row(lp2['eu_b']), row(lp2['eu_ln_g']),
                   row(lp2['eu_ln_b']), p['cls_W1'].T, row(p['cls_b1']),
                   row(p['cls_W2']), p['cls_b2'].reshape(1, 1))


# speedup vs baseline: 6.0022x; 1.0315x over previous
import functools

import jax
import jax.numpy as jnp
from jax.experimental import pallas as pl
from jax.experimental.pallas import tpu as pltpu
from jax.experimental.pallas import tpu_sc as plsc

F32 = jnp.float32
LANES = 128
NW = 32
TE = 2000
TN = 2048


def _leaky(x):
    return jnp.where(x >= 0, x, 0.01 * x)


def _lnorm(x, g, b):
    mu = x.mean(axis=-1, keepdims=True)
    var = ((x - mu) ** 2).mean(axis=-1, keepdims=True)
    return (x - mu) / jnp.sqrt(var + 1e-5) * g + b


def _dot(a, b):
    return jnp.dot(a, b, preferred_element_type=F32)


def _full(shape):
    return pl.BlockSpec(shape, lambda i: (0,) * len(shape))


def _node_encoder(x, wt, b, g, bb):
    n = x.shape[0]

    def body(x_ref, wt_ref, b_ref, g_ref, bb_ref, o_ref):
        y = _dot(x_ref[...], wt_ref[...]) + b_ref[...]
        o_ref[...] = _lnorm(_leaky(y), g_ref[...], bb_ref[...])

    return pl.pallas_call(
        body,
        grid=(n // TN,),
        in_specs=[
            pl.BlockSpec((TN, LANES), lambda i: (i, 0)),
            _full((LANES, LANES)), _full((1, LANES)), _full((1, LANES)),
            _full((1, LANES)),
        ],
        out_specs=pl.BlockSpec((TN, LANES), lambda i: (i, 0)),
        out_shape=jax.ShapeDtypeStruct((n, LANES), F32),
    )(x, wt, b, g, bb)


def _rel_table(relation_feat, w128t, rel_embed):
    nr = relation_feat.shape[0]

    def body(rf_ref, wt_ref, re_ref, o_ref):
        o_ref[:, 0:LANES] = _dot(rf_ref[...], wt_ref[...])
        o_ref[:, LANES:2 * LANES] = re_ref[...]

    return pl.pallas_call(
        body,
        grid=(1,),
        in_specs=[_full((nr, LANES)), _full((LANES, LANES)),
                  _full((nr, LANES))],
        out_specs=_full((nr, 2 * LANES)),
        out_shape=jax.ShapeDtypeStruct((nr, 2 * LANES), F32),
    )(relation_feat, w128t, rel_embed)


def _edge_encoder(t0g, q, wq, b, g, bb):
    m = t0g.shape[0]

    def body(r1_ref, q_ref, wq_ref, b_ref, g_ref, bb_ref, o_ref):
        y = r1_ref[...] + q_ref[...] * wq_ref[...] + b_ref[...]
        o_ref[...] = _lnorm(_leaky(y), g_ref[...], bb_ref[...])

    return pl.pallas_call(
        body,
        grid=(m // TE,),
        in_specs=[
            pl.BlockSpec((TE, LANES), lambda i: (i, 0)),
            pl.BlockSpec((TE, 1), lambda i: (i, 0)),
            _full((1, LANES)), _full((1, LANES)), _full((1, LANES)),
            _full((1, LANES)),
        ],
        out_specs=pl.BlockSpec((TE, LANES), lambda i: (i, 0)),
        out_shape=jax.ShapeDtypeStruct((m, LANES), F32),
    )(t0g, q, wq, b, g, bb)


def _msg(hh, ht, e, t0g, wft, wbt, bf, bb):
    m = e.shape[0]
    L = LANES

    def body(hh_ref, ht_ref, e_ref, r_ref, wf_ref, wb_ref, bf_ref, bb_ref,
             mf_ref, mb_ref):
        ev = e_ref[...]
        rv = r_ref[...]
        er = ev * rv
        wf = wf_ref[...]
        wb = wb_ref[...]
        hhv = hh_ref[...]
        htv = ht_ref[...]
        mf_ref[...] = (_dot(hhv, wf[0:L]) + _dot(ev, wf[L:2 * L])
                       + _dot(hhv * rv, wf[2 * L:3 * L])
                       + _dot(er, wf[3 * L:4 * L]) + bf_ref[...])
        mb_ref[...] = (_dot(htv, wb[0:L]) + _dot(ev, wb[L:2 * L])
                       + _dot(htv * rv, wb[2 * L:3 * L])
                       + _dot(er, wb[3 * L:4 * L]) + bb_ref[...])

    edge = pl.BlockSpec((TE, LANES), lambda i: (i, 0))
    return pl.pallas_call(
        body,
        grid=(m // TE,),
        in_specs=[
            edge, edge, edge,
            pl.BlockSpec((TE, LANES), lambda i: (i, 1)),
            _full((4 * LANES, LANES)), _full((4 * LANES, LANES)),
            _full((1, LANES)), _full((1, LANES)),
        ],
        out_specs=[edge, edge],
        out_shape=[jax.ShapeDtypeStruct((m, LANES), F32),
                   jax.ShapeDtypeStruct((m, LANES), F32)],
    )(hh, ht, e, t0g, wft, wbt, bf, bb)


def _msg_terms(hv, tv, ev, rv, wf, wb, bf, bb):
    L = LANES
    er = ev * rv
    mf = (_dot(hv, wf[0:L]) + _dot(ev, wf[L:2 * L])
          + _dot(hv * rv, wf[2 * L:3 * L]) + _dot(er, wf[3 * L:4 * L]) + bf)
    mb = (_dot(tv, wb[0:L]) + _dot(ev, wb[L:2 * L])
          + _dot(tv * rv, wb[2 * L:3 * L]) + _dot(er, wb[3 * L:4 * L]) + bb)
    return mf, mb


def _enc_msg(t0g, q, wq, be, ge, bbe, hh, ht, wft, wbt, bf, bb):
    m = hh.shape[0]

    def body(t_ref, q_ref, wq_ref, be_ref, ge_ref, bbe_ref, hh_ref, ht_ref,
             wf_ref, wb_ref, bf_ref, bb_ref, e0_ref, mf_ref, mb_ref):
        t = t_ref[...]
        e0 = _lnorm(_leaky(t[:, 0:LANES] + q_ref[...] * wq_ref[...]
                           + be_ref[...]), ge_ref[...], bbe_ref[...])
        e0_ref[...] = e0
        mf, mb = _msg_terms(hh_ref[...], ht_ref[...], e0, t[:, LANES:],
                            wf_ref[...], wb_ref[...], bf_ref[...],
                            bb_ref[...])
        mf_ref[...] = mf
        mb_ref[...] = mb

    edge = pl.BlockSpec((TE, LANES), lambda i: (i, 0))
    return pl.pallas_call(
        body,
        grid=(m // TE,),
        in_specs=[
            pl.BlockSpec((TE, 2 * LANES), lambda i: (i, 0)),
            pl.BlockSpec((TE, 1), lambda i: (i, 0)),
            _full((1, LANES)), _full((1, LANES)), _full((1, LANES)),
            _full((1, LANES)), edge, edge,
            _full((4 * LANES, LANES)), _full((4 * LANES, LANES)),
            _full((1, LANES)), _full((1, LANES)),
        ],
        out_specs=[edge, edge, edge],
        out_shape=[jax.ShapeDtypeStruct((m, LANES), F32)] * 3,
    )(t0g, q, wq, be, ge, bbe, hh, ht, wft, wbt, bf, bb)


def _eu_msg(hh, ht, e, t0g, euwt, eub, eug, eubb, wft, wbt, bf, bb):
    m = e.shape[0]
    L = LANES

    def body(hh_ref, ht_ref, e_ref, r_ref, w_ref, b_ref, g_ref, bb_ref,
             wf_ref, wb_ref, bf_ref, bb2_ref, en_ref, mf_ref, mb_ref):
        ev = e_ref[...]
        hhv = hh_ref[...]
        htv = ht_ref[...]
        w = w_ref[...]
        y = (_dot(hhv, w[0:L]) + _dot(ev, w[L:2 * L])
             + _dot(htv, w[2 * L:3 * L]) + b_ref[...])
        en = _lnorm(_leaky(y) + ev, g_ref[...], bb_ref[...])
        en_ref[...] = en
        mf, mb = _msg_terms(hhv, htv, en, r_ref[...], wf_ref[...],
                            wb_ref[...], bf_ref[...], bb2_ref[...])
        mf_ref[...] = mf
        mb_ref[...] = mb

    edge = pl.BlockSpec((TE, LANES), lambda i: (i, 0))
    return pl.pallas_call(
        body,
        grid=(m // TE,),
        in_specs=[edge, edge, edge,
                  pl.BlockSpec((TE, LANES), lambda i: (i, 1)),
                  _full((3 * LANES, LANES)), _full((1, LANES)),
                  _full((1, LANES)), _full((1, LANES)),
                  _full((4 * LANES, LANES)), _full((4 * LANES, LANES)),
                  _full((1, LANES)), _full((1, LANES))],
        out_specs=[edge, edge, edge],
        out_shape=[jax.ShapeDtypeStruct((m, LANES), F32)] * 3,
    )(hh, ht, e, t0g, euwt, eub, eug, eubb, wft, wbt, bf, bb)


def _eu_cls(hh, ht, e, e0, h0h, h0t, euwt, eub, eug, eubb, w1t, b1, w2, b2):
    m = e.shape[0]
    L = LANES

    def body(hh_ref, ht_ref, e_ref, e0_ref, h0h_ref, h0t_ref, w_ref, b_ref,
             g_ref, bb_ref, w1_ref, b1_ref, w2_ref, b2_ref, o_ref):
        ev = e_ref[...]
        hhv = hh_ref[...]
        htv = ht_ref[...]
        w = w_ref[...]
        y = (_dot(hhv, w[0:L]) + _dot(ev, w[L:2 * L])
             + _dot(htv, w[2 * L:3 * L]) + b_ref[...])
        e2 = _lnorm(_leaky(y) + ev, g_ref[...], bb_ref[...])
        w1 = w1_ref[...]
        y1 = (_dot(e2, w1[0:L]) + _dot(e0_ref[...], w1[L:2 * L])
              + _dot(hhv, w1[2 * L:3 * L])
              + _dot(h0h_ref[...], w1[3 * L:4 * L])
              + _dot(htv, w1[4 * L:5 * L])
              + _dot(h0t_ref[...], w1[5 * L:6 * L]) + b1_ref[...])
        o1 = _leaky(y1)
        o_ref[...] = (jnp.sum(o1 * w2_ref[...], axis=1, keepdims=True)
                      + b2_ref[...])

    edge = pl.BlockSpec((TE, LANES), lambda i: (i, 0))
    return pl.pallas_call(
        body,
        grid=(m // TE,),
        in_specs=[edge, edge, edge, edge, edge, edge,
                  _full((3 * LANES, LANES)), _full((1, LANES)),
                  _full((1, LANES)), _full((1, LANES)),
                  _full((6 * LANES, LANES)), _full((1, LANES)),
                  _full((1, LANES)), _full((1, 1))],
        out_specs=pl.BlockSpec((TE, 1), lambda i: (i, 0)),
        out_shape=jax.ShapeDtypeStruct((m, 1), F32),
    )(hh, ht, e, e0, h0h, h0t, euwt, eub, eug, eubb, w1t, b1, w2, b2)


def _h_update(partials_list, cnt_list, h, g, b):
    n = h.shape[0]
    np_ = len(partials_list)
    ncn = len(cnt_list)

    def body(*refs):
        p_refs = refs[:np_]
        c_refs = refs[np_:np_ + ncn]
        h_ref, g_ref, b_ref, o_ref = refs[np_ + ncn:]
        agg = sum(p[0] + p[1] for p in p_refs)
        cnt = sum(c[0] + c[1] for c in c_refs)[:, None]
        o_ref[...] = _lnorm(_leaky(agg / cnt) + h_ref[...],
                            g_ref[...], b_ref[...])

    return pl.pallas_call(
        body,
        grid=(n // TN,),
        in_specs=(
            [pl.BlockSpec((2, TN, LANES), lambda i: (0, i, 0))] * np_
            + [pl.BlockSpec((2, TN), lambda i: (0, i))] * ncn
            + [pl.BlockSpec((TN, LANES), lambda i: (i, 0)),
               _full((1, LANES)), _full((1, LANES))]
        ),
        out_specs=pl.BlockSpec((TN, LANES), lambda i: (i, 0)),
        out_shape=jax.ShapeDtypeStruct((n, LANES), F32),
    )(*partials_list, *cnt_list, h, g, b)


def _edge_update(hh, ht, e, wt, b, g, bb):
    m = e.shape[0]
    L = LANES

    def body(hh_ref, ht_ref, e_ref, w_ref, b_ref, g_ref, bb_ref, o_ref):
        ev = e_ref[...]
        w = w_ref[...]
        y = (_dot(hh_ref[...], w[0:L]) + _dot(ev, w[L:2 * L])
             + _dot(ht_ref[...], w[2 * L:3 * L]) + b_ref[...])
        o_ref[...] = _lnorm(_leaky(y) + ev, g_ref[...], bb_ref[...])

    edge = pl.BlockSpec((TE, LANES), lambda i: (i, 0))
    return pl.pallas_call(
        body,
        grid=(m // TE,),
        in_specs=[edge, edge, edge, _full((3 * LANES, LANES)),
                  _full((1, LANES)), _full((1, LANES)), _full((1, LANES))],
        out_specs=edge,
        out_shape=jax.ShapeDtypeStruct((m, LANES), F32),
    )(hh, ht, e, wt, b, g, bb)


def _classifier(e2, e0, hh, h0h, htl, h0t, w1t, b1, w2, b2):
    m = e2.shape[0]
    L = LANES

    def body(e2_ref, e0_ref, hh_ref, h0h_ref, ht_ref, h0t_ref,
             w_ref, b1_ref, w2_ref, b2_ref, o_ref):
        w = w_ref[...]
        y = (_dot(e2_ref[...], w[0:L]) + _dot(e0_ref[...], w[L:2 * L])
             + _dot(hh_ref[...], w[2 * L:3 * L])
             + _dot(h0h_ref[...], w[3 * L:4 * L])
             + _dot(ht_ref[...], w[4 * L:5 * L])
             + _dot(h0t_ref[...], w[5 * L:6 * L]) + b1_ref[...])
        o1 = _leaky(y)
        o_ref[...] = (jnp.sum(o1 * w2_ref[...], axis=1, keepdims=True)
                      + b2_ref[...])

    edge = pl.BlockSpec((TE, LANES), lambda i: (i, 0))
    return pl.pallas_call(
        body,
        grid=(m // TE,),
        in_specs=[edge, edge, edge, edge, edge, edge,
                  _full((6 * LANES, LANES)), _full((1, LANES)),
                  _full((1, LANES)), _full((1, 1))],
        out_specs=pl.BlockSpec((TE, 1), lambda i: (i, 0)),
        out_shape=jax.ShapeDtypeStruct((m, 1), F32),
    )(e2, e0, hh, h0h, htl, h0t, w1t, b1, w2, b2)


def _gather_many(table, idxs):
    n, d = table.shape
    r = idxs[0].shape[0] // 128
    b = -(-r // NW)
    nbuf = 3 if d > LANES else 6
    nio = len(idxs)

    def body(tab_hbm, *refs):
        idx_hbms = refs[:nio]
        out_hbms = refs[nio:2 * nio]
        idx_all = refs[2 * nio]
        rows = refs[2 * nio + 1:2 * nio + 1 + nbuf]
        sg = refs[2 * nio + 1 + nbuf:2 * nio + 1 + 2 * nbuf]
        so = refs[2 * nio + 1 + 2 * nbuf:2 * nio + 1 + 3 * nbuf]
        c = jax.lax.axis_index("c")
        s = jax.lax.axis_index("s")
        w = s * 2 + c
        start = w * b
        nc = jnp.clip(r - start, 0, b)
        cp = jnp.minimum(start, r - b)
        off = start - cp

        for idx_hbm, out_hbm in zip(idx_hbms, out_hbms):
            pltpu.sync_copy(idx_hbm.at[pl.ds(cp * 128, b * 128)], idx_all)

            def step(p, carry):
                for u in range(nbuf):
                    j = p * nbuf + u

                    @pl.when(j < nc)
                    def _(j=j, u=u):
                        @pl.when(p > 0)
                        def _():
                            pltpu.make_async_copy(
                                rows[u], out_hbm.at[pl.ds(0, 128)],
                                so[u]).wait()
                        o = pl.multiple_of((off + j) * 128, 128)
                        pltpu.async_copy(
                            tab_hbm.at[idx_all.at[pl.ds(o, 128)]],
                            rows[u], sg[u])
                for u in range(nbuf):
                    j = p * nbuf + u

                    @pl.when(j < nc)
                    def _(j=j, u=u):
                        pltpu.make_async_copy(
                            tab_hbm.at[idx_all.at[pl.ds(0, 128)]],
                            rows[u], sg[u]).wait()
                        pltpu.async_copy(
                            rows[u], out_hbm.at[pl.ds((start + j) * 128, 128)],
                            so[u])
                return carry

            jax.lax.fori_loop(0, -(-b // nbuf), step, 0)
            for u in range(nbuf):
                @pl.when(u < nc)
                def _(u=u):
                    pltpu.make_async_copy(
                        rows[u], out_hbm.at[pl.ds(0, 128)], so[u]).wait()

    f = pl.kernel(
        body,
        out_type=tuple(jax.ShapeDtypeStruct((r * 128, d), F32)
                       for _ in range(nio)),
        mesh=plsc.VectorSubcoreMesh(core_axis_name="c", subcore_axis_name="s"),
        scratch_types=(
            [pltpu.VMEM((b * 128,), jnp.int32)]
            + [pltpu.VMEM((128, d), F32) for _ in range(nbuf)]
            + [pltpu.SemaphoreType.DMA for _ in range(2 * nbuf)]
        ),
    )
    out = f(table, *idxs)
    if not isinstance(out, (list, tuple)):
        out = (out,)
    return tuple(out)


def _scatter_add(mf, mb, idxf2d, idxb2d, n, with_counts):
    r = mf.shape[0] // 128
    b = idxf2d.shape[0] // 16
    nbuf = 2
    npad = ((n + 2047) // 2048) * 2048
    npt = npad // 16

    def body(mf_hbm, mb_hbm, if_hbm, ib_hbm, outp_hbm, outc_hbm,
             tab_sh, cnt_sh, idx_v, ones_v, cbuf_v, *rest):
        rows = rest[:nbuf]
        sm = rest[nbuf:2 * nbuf]
        ss = rest[2 * nbuf:3 * nbuf]
        sc = rest[3 * nbuf]
        c = jax.lax.axis_index("c")
        s = jax.lax.axis_index("s")
        zv = jnp.zeros((16,), F32)
        ov = jnp.ones((16,), F32)

        def zfill(i, carry):
            rows[0][i // 8, pl.ds((i % 8) * 16, 16)] = zv
            return carry
        jax.lax.fori_loop(0, 128 * 8, zfill, 0)

        def ofill(i, carry):
            ones_v[pl.ds(i * 16, 16)] = ov
            cbuf_v[pl.ds(i * 16, 16)] = zv
            return carry
        jax.lax.fori_loop(0, 8, ofill, 0)

        def ztab(z, carry):
            pltpu.sync_copy(rows[0], tab_sh.at[pl.ds(s * npt + z * 128, 128)])
            if with_counts:
                pltpu.sync_copy(cbuf_v,
                                cnt_sh.at[pl.ds(s * npt + z * 128, 128)])
            return carry
        jax.lax.fori_loop(0, npt // 128, ztab, 0)
        plsc.subcore_barrier()

        start = s * b
        nc = jnp.clip(r - start, 0, b)

        for d_id, (msg_hbm, idx_hbm) in enumerate(
                ((mf_hbm, if_hbm), (mb_hbm, ib_hbm))):
            @pl.when(c == d_id)
            def _(msg_hbm=msg_hbm, idx_hbm=idx_hbm):
                pltpu.sync_copy(idx_hbm.at[pl.ds(start, b)], idx_v)

                def step(p, carry):
                    for u in range(nbuf):
                        j = p * nbuf + u

                        @pl.when(j < nc)
                        def _(j=j, u=u):
                            @pl.when(p > 0)
                            def _():
                                pltpu.make_async_copy(
                                    rows[u], tab_sh.at[pl.ds(0, 128)],
                                    ss[u]).wait()
                            pltpu.async_copy(
                                msg_hbm.at[pl.ds((start + j) * 128, 128)],
                                rows[u], sm[u])
                    for u in range(nbuf):
                        j = p * nbuf + u

                        @pl.when(j < nc)
                        def _(j=j, u=u):
                            pltpu.make_async_copy(
                                msg_hbm.at[pl.ds(0, 128)], rows[u],
                                sm[u]).wait()
                            pltpu.async_copy(rows[u], tab_sh.at[idx_v.at[j]],
                                             ss[u], add=True)
                            if with_counts:
                                pltpu.async_copy(ones_v,
                                                 cnt_sh.at[idx_v.at[j]], sc,
                                                 add=True)
                    return carry

                jax.lax.fori_loop(0, -(-b // nbuf), step, 0)
                for u in range(nbuf):
                    @pl.when(u < nc)
                    def _(u=u):
                        pltpu.make_async_copy(
                            rows[u], tab_sh.at[pl.ds(0, 128)], ss[u]).wait()
                if with_counts:
                    def cdrain(j, carry):
                        @pl.when(j < nc)
                        def _():
                            pltpu.make_async_copy(
                                ones_v, cnt_sh.at[pl.ds(0, 128)], sc).wait()
                        return carry
                    jax.lax.fori_loop(0, b, cdrain, 0)
        plsc.subcore_barrier()

        def wout(z, carry):
            a = s * npt + z * 128
            pltpu.sync_copy(tab_sh.at[pl.ds(a, 128)], rows[0])
            pltpu.sync_copy(rows[0], outp_hbm.at[c, pl.ds(a, 128)])
            if with_counts:
                pltpu.sync_copy(cnt_sh.at[pl.ds(a, 128)], cbuf_v)
                pltpu.sync_copy(cbuf_v, outc_hbm.at[c, pl.ds(a, 128)])
            return carry
        jax.lax.fori_loop(0, npt // 128, wout, 0)

    f = pl.kernel(
        body,
        out_type=(jax.ShapeDtypeStruct((2, npad, LANES), F32),
                  jax.ShapeDtypeStruct((2, npad), F32)),
        mesh=plsc.VectorSubcoreMesh(core_axis_name="c", subcore_axis_name="s"),
        scratch_types=(
            [pltpu.VMEM_SHARED((npad, LANES), F32),
             pltpu.VMEM_SHARED((npad,), F32),
             pltpu.VMEM((b, 128), jnp.int32),
             pltpu.VMEM((128,), F32),
             pltpu.VMEM((128,), F32)]
            + [pltpu.VMEM((128, LANES), F32) for _ in range(nbuf)]
            + [pltpu.SemaphoreType.DMA for _ in range(2 * nbuf + 1)]
        ),
    )
    return f(mf, mb, idxf2d, idxb2d)


def kernel(ht, r_tensor, entity_feat, relation_feat, p_selections, queries,
           params):
    del p_selections
    m = ht.shape[0]
    n = entity_feat.shape[0]
    npad = ((n + 2047) // 2048) * 2048
    ef_pad = jnp.pad(entity_feat, ((0, npad - n), (0, 0)))

    hix = ht[:, 0].astype(jnp.int32)
    tix = ht[:, 1].astype(jnp.int32)
    rix = r_tensor.astype(jnp.int32)
    q2d = queries.reshape(m, 1)

    m2 = m // 2
    halves = []
    for lo in (0, m2):
        hx = jax.lax.slice(hix, (lo,), (lo + m2,))
        tx = jax.lax.slice(tix, (lo,), (lo + m2,))
        rx = jax.lax.slice(rix, (lo,), (lo + m2,))
        qd = jax.lax.slice(q2d, (lo, 0), (lo + m2, 1))
        rc = m2 // 128
        bslot = ((-(-rc // 16) + 7) // 8) * 8
        rpad = 16 * bslot
        hx2d = jnp.pad(hx.reshape(rc, 128), ((0, rpad - rc), (0, 0)))
        tx2d = jnp.pad(tx.reshape(rc, 128), ((0, rpad - rc), (0, 0)))
        halves.append(dict(hx=hx, tx=tx, rx=rx, qd=qd, hx2d=hx2d, tx2d=tx2d))

    p = params
    row = lambda v: v.reshape(1, -1)

    h0 = _node_encoder(ef_pad, p['ent_in_W'].T, row(p['ent_in_b']),
                       row(p['ln_ent_g']), row(p['ln_ent_b']))
    t0 = _rel_table(relation_feat, p['edge_in_W'][:, :LANES].T,
                    p['rel_embed'])
    lp1, lp2 = p['layers']

    for hv in halves:
        hv['t0g'], = _gather_many(t0, [hv['rx']])
        hv['h0h'], hv['h0t'] = _gather_many(h0, [hv['hx'], hv['tx']])

    p1, c1 = [], []
    for hv in halves:
        hv['e0'], mf1, mb1 = _enc_msg(
            hv['t0g'], hv['qd'], row(p['edge_in_W'][:, LANES]),
            row(p['edge_in_b']), row(p['ln_edge_g']), row(p['ln_edge_b']),
            hv['h0h'], hv['h0t'], lp1['fwd_W'].T, lp1['back_W'].T,
            row(lp1['fwd_b']), row(lp1['back_b']))
        pp, cc = _scatter_add(mf1, mb1, hv['tx2d'], hv['hx2d'], npad,
                              with_counts=True)
        p1.append(pp)
        c1.append(cc)
    h1 = _h_update(p1, c1, h0, row(lp1['mp_ln_g']), row(lp1['mp_ln_b']))

    p2 = []
    for hv in halves:
        hv['h1h'], hv['h1t'] = _gather_many(h1, [hv['hx'], hv['tx']])
    for hv in halves:
        hv['e1'], mf2, mb2 = _eu_msg(
            hv['h1h'], hv['h1t'], hv['e0'], hv['t0g'], lp1['eu_W'].T,
            row(lp1['eu_b']), row(lp1['eu_ln_g']), row(lp1['eu_ln_b']),
            lp2['fwd_W'].T, lp2['back_W'].T, row(lp2['fwd_b']),
            row(lp2['back_b']))
        pp, _ = _scatter_add(mf2, mb2, hv['tx2d'], hv['hx2d'], npad,
                             with_counts=False)
        p2.append(pp)
    h2 = _h_update(p2, c1, h1, row(lp2['mp_ln_g']), row(lp2['mp_ln_b']))

    outs = []
    for hv in halves:
        h2h, h2t = _gather_many(h2, [hv['hx'], hv['tx']])
        outs.append(_eu_cls(
            h2h, h2t, hv['e1'], hv['e0'], hv['h0h'], hv['h0t'],
            lp2['eu_W'].T, row(lp2['eu_b']), row(lp2['eu_ln_g']),
            row(lp2['eu_ln_b']), p['cls_W1'].T, row(p['cls_b1']),
            row(p['cls_W2']), p['cls_b2'].reshape(1, 1)))
    return jnp.concatenate(outs, axis=0)
